# baseline (speedup 1.0000x reference)
import functools

import jax
import jax.numpy as jnp
from jax import lax
from jax.experimental import pallas as pl
from jax.experimental.pallas import tpu as pltpu
from jax.experimental.pallas import tpu_sc as plsc

N = 10000
NP = 10240
F = 256
FH = 128
E = 160000
P = 8000
NB = 10
A = 1000
R = NB * P
LOW, HIGH = 0.0, 480.0

NC, NS = 2, 16
NW = NC * NS
CH = 128
CP = 64
E2 = 163840
EC2 = E2 // CH
ECT_A = EC2 // NW
ECT_B = EC2 // NS
RP = 81920
RC2 = RP // CP
RCT = RC2 // NW
ROWS_PER_TILE = NP // NS
HN = NP // 2
HNP = HN + 128
ROWS_PER_TILE_A = HNP // NS

_mesh = plsc.VectorSubcoreMesh(core_axis_name="c", subcore_axis_name="s")


@functools.partial(
    pl.kernel, mesh=_mesh,
    out_type=jax.ShapeDtypeStruct((NC, HNP, FH), jnp.float32),
    scratch_types=[pltpu.VMEM((CH,), jnp.int32),
                   pltpu.VMEM((CH,), jnp.int32),
                   pltpu.VMEM((CH,), jnp.int32),
                   pltpu.VMEM((CH,), jnp.int32),
                   pltpu.VMEM((CH, FH), jnp.float32),
                   pltpu.VMEM_SHARED((HNP, FH), jnp.float32),
                   pltpu.SemaphoreType.DMA,
                   pltpu.SemaphoreType.DMA,
                   pltpu.SemaphoreType.DMA,
                   pltpu.SemaphoreType.DMA,
                   pltpu.SemaphoreType.DMA,
                   pltpu.SemaphoreType.DMA,
                   pltpu.SemaphoreType.DMA,
                   pltpu.SemaphoreType.DMA],
)
def _sc_deg(dstc_hbm, ones_hbm, zeros_hbm, out_hbm,
            di0, di1, di2, di3, ones_v, acc_sh,
            is0, is1, is2, is3, s0, s1, s2, s3):
    c = lax.axis_index("c")
    s = lax.axis_index("s")
    pltpu.sync_copy(ones_hbm, ones_v)
    sl = pl.ds(s * ROWS_PER_TILE_A, ROWS_PER_TILE_A)
    pltpu.sync_copy(zeros_hbm.at[pl.ds(0, ROWS_PER_TILE_A)], acc_sh.at[sl])
    plsc.subcore_barrier()
    dis = [di0, di1, di2, di3]
    isems = [is0, is1, is2, is3]
    ssems = [s0, s1, s2, s3]
    src = dstc_hbm.at[c]

    @pl.loop(0, ECT_B, step=4)
    def _(k):
        icps = []
        for b in range(4):
            rows = pl.ds(((k + b) * NS + s) * CH, CH)
            icps.append(pltpu.async_copy(src.at[rows], dis[b], isems[b]))
        scps = []
        for b in range(4):
            icps[b].wait()
            scps.append(pltpu.async_copy(ones_v, acc_sh.at[dis[b]],
                                         ssems[b], add=True))
        for b in range(4):
            scps[b].wait()

    plsc.subcore_barrier()
    pltpu.sync_copy(acc_sh.at[sl], out_hbm.at[c].at[sl])


@functools.partial(
    pl.kernel, mesh=_mesh,
    out_type=jax.ShapeDtypeStruct((NC, NP, FH), jnp.float32),
    scratch_types=[pltpu.VMEM((CH,), jnp.int32),
                   pltpu.VMEM((CH,), jnp.int32),
                   pltpu.VMEM((CH,), jnp.int32),
                   pltpu.VMEM((CH,), jnp.int32),
                   pltpu.VMEM((CH, FH), jnp.float32),
                   pltpu.VMEM((CH, FH), jnp.float32),
                   pltpu.VMEM_SHARED((NP, FH), jnp.float32),
                   pltpu.SemaphoreType.DMA,
                   pltpu.SemaphoreType.DMA],
)
def _sc_gcn_agg(src2_hbm, dst2_hbm, xsp_hbm, zeros_hbm, out_hbm,
                si0, di0, si1, di1, rb0, rb1, acc_sh, g0, g1):
    c = lax.axis_index("c")
    s = lax.axis_index("s")
    sl = pl.ds(s * ROWS_PER_TILE, ROWS_PER_TILE)
    pltpu.sync_copy(zeros_hbm, acc_sh.at[sl])
    plsc.subcore_barrier()
    tbl = xsp_hbm.at[c]

    @pl.loop(0, ECT_B, step=2)
    def _(k):
        r0 = pl.ds((k * NS + s) * CH, CH)
        r1 = pl.ds(((k + 1) * NS + s) * CH, CH)
        pltpu.sync_copy(src2_hbm.at[r0], si0)
        pltpu.sync_copy(dst2_hbm.at[r0], di0)
        cp0 = pltpu.async_copy(tbl.at[si0], rb0, g0)
        pltpu.sync_copy(src2_hbm.at[r1], si1)
        pltpu.sync_copy(dst2_hbm.at[r1], di1)
        cp1 = pltpu.async_copy(tbl.at[si1], rb1, g1)
        cp0.wait()
        pltpu.sync_copy(rb0, acc_sh.at[di0], add=True)
        cp1.wait()
        pltpu.sync_copy(rb1, acc_sh.at[di1], add=True)

    plsc.subcore_barrier()
    pltpu.sync_copy(acc_sh.at[sl], out_hbm.at[c].at[sl])


@functools.partial(
    pl.kernel, mesh=_mesh,
    out_type=jax.ShapeDtypeStruct((2, RP, F), jnp.float32),
    scratch_types=[pltpu.VMEM((CP,), jnp.int32),
                   pltpu.VMEM((CP,), jnp.int32),
                   pltpu.VMEM((CP,), jnp.int32),
                   pltpu.VMEM((CP,), jnp.int32),
                   pltpu.VMEM((CP, F), jnp.float32),
                   pltpu.VMEM((CP, F), jnp.float32),
                   pltpu.VMEM((CP, F), jnp.float32),
                   pltpu.VMEM((CP, F), jnp.float32),
                   pltpu.SemaphoreType.DMA,
                   pltpu.SemaphoreType.DMA,
                   pltpu.SemaphoreType.DMA,
                   pltpu.SemaphoreType.DMA,
                   pltpu.SemaphoreType.DMA,
                   pltpu.SemaphoreType.DMA,
                   pltpu.SemaphoreType.DMA,
                   pltpu.SemaphoreType.DMA],
)
def _sc_pair_gather(xs2_hbm, xd2_hbm, i02_hbm, i12_hbm, out_hbm,
                    i0a, i1a, i0b, i1b, ra0, rb0, ra1, rb1,
                    is0, is1, is2, is3, g0, g1, g2, g3):
    c = lax.axis_index("c")
    s = lax.axis_index("s")
    wid = s * NC + c

    @pl.loop(0, RCT, step=2)
    def _(k):
        j0 = wid * RCT + k
        r0 = pl.ds(j0 * CP, CP)
        r1 = pl.ds((j0 + 1) * CP, CP)
        ic0 = pltpu.async_copy(i02_hbm.at[r0], i0a, is0)
        ic1 = pltpu.async_copy(i12_hbm.at[r0], i1a, is1)
        ic2 = pltpu.async_copy(i02_hbm.at[r1], i0b, is2)
        ic3 = pltpu.async_copy(i12_hbm.at[r1], i1b, is3)
        ic0.wait()
        cp0 = pltpu.async_copy(xs2_hbm.at[i0a], ra0, g0)
        ic1.wait()
        cp1 = pltpu.async_copy(xd2_hbm.at[i1a], rb0, g1)
        ic2.wait()
        cp2 = pltpu.async_copy(xs2_hbm.at[i0b], ra1, g2)
        ic3.wait()
        cp3 = pltpu.async_copy(xd2_hbm.at[i1b], rb1, g3)
        cp0.wait()
        pltpu.sync_copy(ra0, out_hbm.at[0].at[r0])
        cp1.wait()
        pltpu.sync_copy(rb0, out_hbm.at[1].at[r0])
        cp2.wait()
        pltpu.sync_copy(ra1, out_hbm.at[0].at[r1])
        cp3.wait()
        pltpu.sync_copy(rb1, out_hbm.at[1].at[r1])


def _tc_xw(state, Wg):
    BLK = 1000

    def body(st_ref, wg_ref, o_ref):
        o_ref[...] = jnp.dot(st_ref[...], wg_ref[...],
                             preferred_element_type=jnp.float32)

    return pl.pallas_call(
        body,
        grid=(N // BLK,),
        in_specs=[pl.BlockSpec((BLK, F), lambda i: (i, 0)),
                  pl.BlockSpec((F, F), lambda i: (0, 0))],
        out_specs=pl.BlockSpec((BLK, F), lambda i: (i, 0)),
        out_shape=jax.ShapeDtypeStruct((N, F), jnp.float32),
    )(state, Wg)


def _tc_scale_split(xw, degp):
    BLK = 2000

    def body(xw_ref, dg_ref, out_ref):
        deg = dg_ref[...][:, 0:1] + 1.0
        dinv = lax.rsqrt(deg)
        xs = xw_ref[...] * dinv
        out_ref[0] = xs[:, :FH]
        out_ref[1] = xs[:, FH:]

    return pl.pallas_call(
        body,
        grid=(N // BLK,),
        in_specs=[pl.BlockSpec((BLK, F), lambda i: (i, 0)),
                  pl.BlockSpec((BLK, 8), lambda i: (i, 0))],
        out_specs=pl.BlockSpec((NC, BLK, FH), lambda i: (0, i, 0)),
        out_shape=jax.ShapeDtypeStruct((NC, NP, FH), jnp.float32),
    )(xw, degp)


def _tc_node_mlp_in(accp, xsp, degp, state, bg, W1t, W1b):
    BLK = 1000

    def body(ac_ref, xs_ref, dg_ref, st_ref, bg_ref, w1t_ref, w1b_ref,
             o1_ref, o2_ref):
        acc = jnp.concatenate([ac_ref[0], ac_ref[1]], axis=1)
        xs = jnp.concatenate([xs_ref[0], xs_ref[1]], axis=1)
        deg = dg_ref[...][:, 0:1] + 1.0
        dinv = lax.rsqrt(deg)
        gcn = (acc + xs) * dinv + bg_ref[...]
        x = jnp.maximum(gcn, 0.0) + st_ref[...]
        o1_ref[...] = jnp.dot(x, w1t_ref[...], preferred_element_type=jnp.float32)
        o2_ref[...] = jnp.dot(x, w1b_ref[...], preferred_element_type=jnp.float32)

    return pl.pallas_call(
        body,
        grid=(N // BLK,),
        in_specs=[pl.BlockSpec((NC, BLK, FH), lambda i: (0, i, 0)),
                  pl.BlockSpec((NC, BLK, FH), lambda i: (0, i, 0)),
                  pl.BlockSpec((BLK, 8), lambda i: (i, 0)),
                  pl.BlockSpec((BLK, F), lambda i: (i, 0)),
                  pl.BlockSpec((F,), lambda i: (0,)),
                  pl.BlockSpec((F, F), lambda i: (0, 0)),
                  pl.BlockSpec((F, F), lambda i: (0, 0))],
        out_specs=[pl.BlockSpec((BLK, F), lambda i: (i, 0)),
                   pl.BlockSpec((BLK, F), lambda i: (i, 0))],
        out_shape=[jax.ShapeDtypeStruct((N, F), jnp.float32),
                   jax.ShapeDtypeStruct((N, F), jnp.float32)],
    )(accp, xsp, degp, state, bg, W1t, W1b)


def _tc_head(g2, b1, W2, b2, Wmu, bmu):
    BLK = 3200

    def body(g_ref, b1_ref, w2_ref, b2_ref, wmu_ref, bmu_ref, o_ref):
        h = g_ref[0] + g_ref[1] + b1_ref[...]
        h = jnp.where(h > 0, h, 0.01 * h)
        h2 = jnp.dot(h, w2_ref[...], preferred_element_type=jnp.float32) + b2_ref[...]
        h2 = jnp.where(h2 > 0, h2, 0.01 * h2)
        m = jnp.dot(h2, wmu_ref[...], preferred_element_type=jnp.float32) + bmu_ref[...]
        mu = jax.nn.softplus(m)
        act = (jnp.tanh(mu) + 1.0) * (0.5 * (HIGH - LOW)) + LOW
        o_ref[...] = jnp.clip(act, LOW, HIGH)

    return pl.pallas_call(
        body,
        grid=(R // BLK,),
        in_specs=[pl.BlockSpec((2, BLK, F), lambda i: (0, i, 0)),
                  pl.BlockSpec((F,), lambda i: (0,)),
                  pl.BlockSpec((F, F), lambda i: (0, 0)),
                  pl.BlockSpec((F,), lambda i: (0,)),
                  pl.BlockSpec((F, 1), lambda i: (0, 0)),
                  pl.BlockSpec((1,), lambda i: (0,))],
        out_specs=pl.BlockSpec((BLK, 1), lambda i: (i, 0)),
        out_shape=jax.ShapeDtypeStruct((R, 1), jnp.float32),
    )(g2, b1, W2, b2, Wmu, bmu)


def kernel(state, edge_index, edges, deterministic,
           Wg, bg, W1, b1, W2, b2, Wmu, bmu, Wsig, bsig):
    del deterministic, Wsig, bsig
    pad = E2 - E
    src2 = jnp.concatenate([edge_index[0], jnp.zeros((pad,), jnp.int32)])
    dst2 = jnp.concatenate([edge_index[1], jnp.full((pad,), N, jnp.int32)])
    onesH = jnp.ones((CH, FH), jnp.float32)
    zerosH = jnp.zeros((ROWS_PER_TILE, FH), jnp.float32)

    dstc = jnp.stack([
        jnp.where(dst2 < HN, dst2, HN),
        jnp.where(dst2 >= HN, dst2 - HN, HN),
    ]).reshape(NC, E2)
    degp = _sc_deg(dstc, onesH, zerosH)
    degc = jnp.concatenate([degp[0, :HN, :8], degp[1, :HN, :8]], axis=0)
    xw = _tc_xw(state, Wg)
    xsp = _tc_scale_split(xw, degc)
    accp = _sc_gcn_agg(src2, dst2, xsp, zerosH)
    xs2, xd2 = _tc_node_mlp_in(accp, xsp, degc, state, bg,
                               W1[:F, :], W1[F:, :])

    boff = (jnp.arange(NB, dtype=jnp.int32) * A)[:, None]
    rpad = jnp.zeros((RP - R,), jnp.int32)
    i0 = jnp.concatenate([(boff + edges[:, 0][None, :]).reshape(-1), rpad])
    i1 = jnp.concatenate([(boff + edges[:, 1][None, :]).reshape(-1), rpad])
    g2 = _sc_pair_gather(xs2, xd2, i0, i1)

    act = _tc_head(g2, b1, W2, b2, Wmu, bmu)
    return act.reshape(NB, P)


# speedup vs baseline: 1.0519x; 1.0519x over previous
import functools

import jax
import jax.numpy as jnp
from jax import lax
from jax.experimental import pallas as pl
from jax.experimental.pallas import tpu as pltpu
from jax.experimental.pallas import tpu_sc as plsc

N = 10000
NP = 10240
F = 256
FH = 128
E = 160000
P = 8000
NB = 10
A = 1000
R = NB * P
LOW, HIGH = 0.0, 480.0

NC, NS = 2, 16
NW = NC * NS
CH = 128
CP = 128
E2 = 163840
EC2 = E2 // CH
ECT_B = EC2 // NS
RP = 81920
RC2 = RP // CP
RCT = RC2 // NW
ROWS_PER_TILE = NP // NS

_mesh = plsc.VectorSubcoreMesh(core_axis_name="c", subcore_axis_name="s")


@functools.partial(
    pl.kernel, mesh=_mesh,
    out_type=jax.ShapeDtypeStruct((NC, NP, FH), jnp.float32),
    scratch_types=[pltpu.VMEM((CH,), jnp.int32),
                   pltpu.VMEM((CH, FH), jnp.float32),
                   pltpu.VMEM_SHARED((NP, FH), jnp.float32),
                   pltpu.SemaphoreType.DMA],
)
def _sc_deg(dst2_hbm, ones_hbm, zeros_hbm, out_hbm, idx_v, ones_v, acc_sh, sem):
    c = lax.axis_index("c")
    s = lax.axis_index("s")
    wid = s * NC + c
    pltpu.sync_copy(ones_hbm, ones_v)
    sl = pl.ds(s * ROWS_PER_TILE, ROWS_PER_TILE)
    pltpu.sync_copy(zeros_hbm, acc_sh.at[sl])
    plsc.subcore_barrier()

    @pl.loop(wid, EC2, step=NW)
    def _(j):
        pltpu.sync_copy(dst2_hbm.at[pl.ds(j * CH, CH)], idx_v)
        pltpu.sync_copy(ones_v, acc_sh.at[idx_v], add=True)

    plsc.subcore_barrier()
    pltpu.sync_copy(acc_sh.at[sl], out_hbm.at[c].at[sl])


@functools.partial(
    pl.kernel, mesh=_mesh,
    out_type=jax.ShapeDtypeStruct((NC, NP, FH), jnp.float32),
    scratch_types=[pltpu.VMEM((CH,), jnp.int32),
                   pltpu.VMEM((CH,), jnp.int32),
                   pltpu.VMEM((CH,), jnp.int32),
                   pltpu.VMEM((CH,), jnp.int32),
                   pltpu.VMEM((CH, FH), jnp.float32),
                   pltpu.VMEM((CH, FH), jnp.float32),
                   pltpu.VMEM_SHARED((NP, FH), jnp.float32),
                   pltpu.SemaphoreType.DMA,
                   pltpu.SemaphoreType.DMA],
)
def _sc_gcn_agg(src2_hbm, dst2_hbm, xsp_hbm, zeros_hbm, out_hbm,
                si0, di0, si1, di1, rb0, rb1, acc_sh, g0, g1):
    c = lax.axis_index("c")
    s = lax.axis_index("s")
    sl = pl.ds(s * ROWS_PER_TILE, ROWS_PER_TILE)
    pltpu.sync_copy(zeros_hbm, acc_sh.at[sl])
    plsc.subcore_barrier()
    tbl = xsp_hbm.at[c]

    def chunk(k):
        return pl.ds((k * NS + s) * CH, CH)

    pltpu.sync_copy(src2_hbm.at[chunk(0)], si0)
    pltpu.sync_copy(dst2_hbm.at[chunk(0)], di0)
    pltpu.async_copy(tbl.at[si0], rb0, g0)

    @pl.loop(0, ECT_B, step=2)
    def _(k):
        pltpu.sync_copy(src2_hbm.at[chunk(k + 1)], si1)
        pltpu.sync_copy(dst2_hbm.at[chunk(k + 1)], di1)
        pltpu.make_async_copy(tbl.at[si0], rb0, g0).wait()
        pltpu.async_copy(tbl.at[si1], rb1, g1)
        pltpu.sync_copy(rb0, acc_sh.at[di0], add=True)

        @pl.when(k + 2 < ECT_B)
        def _():
            pltpu.sync_copy(src2_hbm.at[chunk(k + 2)], si0)
            pltpu.sync_copy(dst2_hbm.at[chunk(k + 2)], di0)

        pltpu.make_async_copy(tbl.at[si1], rb1, g1).wait()

        @pl.when(k + 2 < ECT_B)
        def _():
            pltpu.async_copy(tbl.at[si0], rb0, g0)

        pltpu.sync_copy(rb1, acc_sh.at[di1], add=True)

    plsc.subcore_barrier()
    pltpu.sync_copy(acc_sh.at[sl], out_hbm.at[c].at[sl])


@functools.partial(
    pl.kernel, mesh=_mesh,
    out_type=jax.ShapeDtypeStruct((2, RP, F), jnp.float32),
    scratch_types=[pltpu.VMEM((CP,), jnp.int32),
                   pltpu.VMEM((CP,), jnp.int32),
                   pltpu.VMEM((CP, F), jnp.float32),
                   pltpu.VMEM((CP, F), jnp.float32),
                   pltpu.SemaphoreType.DMA,
                   pltpu.SemaphoreType.DMA],
)
def _sc_pair_gather(xs2_hbm, xd2_hbm, i02_hbm, i12_hbm, out_hbm,
                    i0v, i1v, ra, rb, g0, g1):
    c = lax.axis_index("c")
    s = lax.axis_index("s")
    wid = s * NC + c

    @pl.loop(0, RCT)
    def _(k):
        j = wid * RCT + k
        rows = pl.ds(j * CP, CP)
        pltpu.sync_copy(i02_hbm.at[rows], i0v)
        pltpu.sync_copy(i12_hbm.at[rows], i1v)
        cp0 = pltpu.async_copy(xs2_hbm.at[i0v], ra, g0)
        cp0.wait()
        cp1 = pltpu.async_copy(xd2_hbm.at[i1v], rb, g1)
        pltpu.sync_copy(ra, out_hbm.at[0].at[rows])
        cp1.wait()
        pltpu.sync_copy(rb, out_hbm.at[1].at[rows])


def _tc_scale_split(state, Wg, degp):
    BLK = 1000

    def body(st_ref, wg_ref, dg_ref, out_ref):
        xw = jnp.dot(st_ref[...], wg_ref[...], preferred_element_type=jnp.float32)
        deg = dg_ref[0][:, 0:1] + dg_ref[1][:, 0:1] + 1.0
        dinv = lax.rsqrt(deg)
        xs = xw * dinv
        out_ref[0] = xs[:, :FH]
        out_ref[1] = xs[:, FH:]

    return pl.pallas_call(
        body,
        grid=(N // BLK,),
        in_specs=[pl.BlockSpec((BLK, F), lambda i: (i, 0)),
                  pl.BlockSpec((F, F), lambda i: (0, 0)),
                  pl.BlockSpec((NC, BLK, FH), lambda i: (0, i, 0))],
        out_specs=pl.BlockSpec((NC, BLK, FH), lambda i: (0, i, 0)),
        out_shape=jax.ShapeDtypeStruct((NC, NP, FH), jnp.float32),
    )(state, Wg, degp)


def _tc_node_mlp_in(accp, xsp, degp, state, bg, W1t, W1b):
    BLK = 1000

    def body(ac_ref, xs_ref, dg_ref, st_ref, bg_ref, w1t_ref, w1b_ref,
             o1_ref, o2_ref):
        acc = jnp.concatenate([ac_ref[0], ac_ref[1]], axis=1)
        xs = jnp.concatenate([xs_ref[0], xs_ref[1]], axis=1)
        deg = dg_ref[0][:, 0:1] + dg_ref[1][:, 0:1] + 1.0
        dinv = lax.rsqrt(deg)
        gcn = (acc + xs) * dinv + bg_ref[...]
        x = jnp.maximum(gcn, 0.0) + st_ref[...]
        o1_ref[...] = jnp.dot(x, w1t_ref[...], preferred_element_type=jnp.float32)
        o2_ref[...] = jnp.dot(x, w1b_ref[...], preferred_element_type=jnp.float32)

    return pl.pallas_call(
        body,
        grid=(N // BLK,),
        in_specs=[pl.BlockSpec((NC, BLK, FH), lambda i: (0, i, 0)),
                  pl.BlockSpec((NC, BLK, FH), lambda i: (0, i, 0)),
                  pl.BlockSpec((NC, BLK, FH), lambda i: (0, i, 0)),
                  pl.BlockSpec((BLK, F), lambda i: (i, 0)),
                  pl.BlockSpec((F,), lambda i: (0,)),
                  pl.BlockSpec((F, F), lambda i: (0, 0)),
                  pl.BlockSpec((F, F), lambda i: (0, 0))],
        out_specs=[pl.BlockSpec((BLK, F), lambda i: (i, 0)),
                   pl.BlockSpec((BLK, F), lambda i: (i, 0))],
        out_shape=[jax.ShapeDtypeStruct((N, F), jnp.float32),
                   jax.ShapeDtypeStruct((N, F), jnp.float32)],
    )(accp, xsp, degp, state, bg, W1t, W1b)


def _tc_head(g2, b1, W2, b2, Wmu, bmu):
    BLK = 3200

    def body(g_ref, b1_ref, w2_ref, b2_ref, wmu_ref, bmu_ref, o_ref):
        h = g_ref[0] + g_ref[1] + b1_ref[...]
        h = jnp.where(h > 0, h, 0.01 * h)
        h2 = jnp.dot(h, w2_ref[...], preferred_element_type=jnp.float32) + b2_ref[...]
        h2 = jnp.where(h2 > 0, h2, 0.01 * h2)
        m = jnp.dot(h2, wmu_ref[...], preferred_element_type=jnp.float32) + bmu_ref[...]
        mu = jax.nn.softplus(m)
        act = (jnp.tanh(mu) + 1.0) * (0.5 * (HIGH - LOW)) + LOW
        o_ref[...] = jnp.clip(act, LOW, HIGH)

    return pl.pallas_call(
        body,
        grid=(R // BLK,),
        in_specs=[pl.BlockSpec((2, BLK, F), lambda i: (0, i, 0)),
                  pl.BlockSpec((F,), lambda i: (0,)),
                  pl.BlockSpec((F, F), lambda i: (0, 0)),
                  pl.BlockSpec((F,), lambda i: (0,)),
                  pl.BlockSpec((F, 1), lambda i: (0, 0)),
                  pl.BlockSpec((1,), lambda i: (0,))],
        out_specs=pl.BlockSpec((BLK, 1), lambda i: (i, 0)),
        out_shape=jax.ShapeDtypeStruct((R, 1), jnp.float32),
    )(g2, b1, W2, b2, Wmu, bmu)


def kernel(state, edge_index, edges, deterministic,
           Wg, bg, W1, b1, W2, b2, Wmu, bmu, Wsig, bsig):
    del deterministic, Wsig, bsig
    pad = E2 - E
    src2 = jnp.concatenate([edge_index[0], jnp.zeros((pad,), jnp.int32)])
    dst2 = jnp.concatenate([edge_index[1], jnp.full((pad,), N, jnp.int32)])
    onesH = jnp.ones((CH, FH), jnp.float32)
    zerosH = jnp.zeros((ROWS_PER_TILE, FH), jnp.float32)

    degp = _sc_deg(dst2, onesH, zerosH)
    xsp = _tc_scale_split(state, Wg, degp)
    accp = _sc_gcn_agg(src2, dst2, xsp, zerosH)
    xs2, xd2 = _tc_node_mlp_in(accp, xsp, degp, state, bg,
                               W1[:F, :], W1[F:, :])

    boff = (jnp.arange(NB, dtype=jnp.int32) * A)[:, None]
    rpad = jnp.zeros((RP - R,), jnp.int32)
    i0 = jnp.concatenate([(boff + edges[:, 0][None, :]).reshape(-1), rpad])
    i1 = jnp.concatenate([(boff + edges[:, 1][None, :]).reshape(-1), rpad])
    g2 = _sc_pair_gather(xs2, xd2, i0, i1)

    act = _tc_head(g2, b1, W2, b2, Wmu, bmu)
    return act.reshape(NB, P)


# speedup vs baseline: 1.1329x; 1.0770x over previous
import functools

import jax
import jax.numpy as jnp
from jax import lax
from jax.experimental import pallas as pl
from jax.experimental.pallas import tpu as pltpu
from jax.experimental.pallas import tpu_sc as plsc

N = 10000
NP = 10240
F = 256
FH = 128
E = 160000
P = 8000
NB = 10
A = 1000
R = NB * P
LOW, HIGH = 0.0, 480.0

NC, NS = 2, 16
NW = NC * NS
CH = 128
CP = 128
E2 = 163840
EC2 = E2 // CH
ECT_B = EC2 // NS
RP = 81920
RC2 = RP // CP
RCT = RC2 // NW
ROWS_PER_TILE = NP // NS

_mesh = plsc.VectorSubcoreMesh(core_axis_name="c", subcore_axis_name="s")


@functools.partial(
    pl.kernel, mesh=_mesh,
    out_type=jax.ShapeDtypeStruct((NC, NP, FH), jnp.float32),
    scratch_types=[pltpu.VMEM((CH,), jnp.int32),
                   pltpu.VMEM((CH, FH), jnp.float32),
                   pltpu.VMEM_SHARED((NP, FH), jnp.float32),
                   pltpu.SemaphoreType.DMA],
)
def _sc_deg(dst2_hbm, ones_hbm, zeros_hbm, out_hbm, idx_v, ones_v, acc_sh, sem):
    c = lax.axis_index("c")
    s = lax.axis_index("s")
    wid = s * NC + c
    pltpu.sync_copy(ones_hbm, ones_v)
    sl = pl.ds(s * ROWS_PER_TILE, ROWS_PER_TILE)
    pltpu.sync_copy(zeros_hbm, acc_sh.at[sl])
    plsc.subcore_barrier()

    @pl.loop(wid, EC2, step=NW)
    def _(j):
        pltpu.sync_copy(dst2_hbm.at[pl.ds(j * CH, CH)], idx_v)
        pltpu.sync_copy(ones_v, acc_sh.at[idx_v], add=True)

    plsc.subcore_barrier()
    pltpu.sync_copy(acc_sh.at[sl], out_hbm.at[c].at[sl])


@functools.partial(
    pl.kernel, mesh=_mesh,
    out_type=jax.ShapeDtypeStruct((NC, NP, FH), jnp.float32),
    scratch_types=[pltpu.VMEM((CH,), jnp.int32),
                   pltpu.VMEM((CH,), jnp.int32),
                   pltpu.VMEM((CH,), jnp.int32),
                   pltpu.VMEM((CH,), jnp.int32),
                   pltpu.VMEM((CH, FH), jnp.float32),
                   pltpu.VMEM((CH, FH), jnp.float32),
                   pltpu.VMEM_SHARED((NP, FH), jnp.float32),
                   pltpu.SemaphoreType.DMA,
                   pltpu.SemaphoreType.DMA],
)
def _sc_gcn_agg(src2_hbm, dst2_hbm, xsp_hbm, zeros_hbm, out_hbm,
                si0, di0, si1, di1, rb0, rb1, acc_sh, g0, g1):
    c = lax.axis_index("c")
    s = lax.axis_index("s")
    sl = pl.ds(s * ROWS_PER_TILE, ROWS_PER_TILE)
    pltpu.sync_copy(zeros_hbm, acc_sh.at[sl])
    plsc.subcore_barrier()
    tbl = xsp_hbm.at[c]

    @pl.loop(s, EC2, step=NS)
    def _(j):
        rows = pl.ds(j * CH, CH)
        pltpu.sync_copy(src2_hbm.at[rows], si0)
        pltpu.sync_copy(dst2_hbm.at[rows], di0)
        pltpu.async_copy(tbl.at[si0], rb0, g0).wait()
        pltpu.sync_copy(rb0, acc_sh.at[di0], add=True)

    plsc.subcore_barrier()
    pltpu.sync_copy(acc_sh.at[sl], out_hbm.at[c].at[sl])


@functools.partial(
    pl.kernel, mesh=_mesh,
    out_type=jax.ShapeDtypeStruct((2, RP, F), jnp.float32),
    scratch_types=[pltpu.VMEM((CP,), jnp.int32),
                   pltpu.VMEM((CP,), jnp.int32),
                   pltpu.VMEM((CP, F), jnp.float32),
                   pltpu.VMEM((CP, F), jnp.float32),
                   pltpu.SemaphoreType.DMA,
                   pltpu.SemaphoreType.DMA],
)
def _sc_pair_gather(xs2_hbm, xd2_hbm, i02_hbm, i12_hbm, out_hbm,
                    i0v, i1v, ra, rb, g0, g1):
    c = lax.axis_index("c")
    s = lax.axis_index("s")
    wid = s * NC + c

    @pl.loop(wid, RC2, step=NW)
    def _(j):
        rows = pl.ds(j * CP, CP)
        pltpu.sync_copy(i02_hbm.at[rows], i0v)
        pltpu.sync_copy(i12_hbm.at[rows], i1v)
        cp0 = pltpu.async_copy(xs2_hbm.at[i0v], ra, g0)
        cp1 = pltpu.async_copy(xd2_hbm.at[i1v], rb, g1)
        cp0.wait()
        cp1.wait()
        pltpu.sync_copy(ra, out_hbm.at[0].at[rows])
        pltpu.sync_copy(rb, out_hbm.at[1].at[rows])


def _tc_scale_split(state, Wg, degp):
    BLK = 1000

    def body(st_ref, wg_ref, dg_ref, out_ref):
        xw = jnp.dot(st_ref[...], wg_ref[...], preferred_element_type=jnp.float32)
        deg = dg_ref[0][:, 0:1] + dg_ref[1][:, 0:1] + 1.0
        dinv = lax.rsqrt(deg)
        xs = xw * dinv
        out_ref[0] = xs[:, :FH]
        out_ref[1] = xs[:, FH:]

    return pl.pallas_call(
        body,
        grid=(N // BLK,),
        in_specs=[pl.BlockSpec((BLK, F), lambda i: (i, 0)),
                  pl.BlockSpec((F, F), lambda i: (0, 0)),
                  pl.BlockSpec((NC, BLK, FH), lambda i: (0, i, 0))],
        out_specs=pl.BlockSpec((NC, BLK, FH), lambda i: (0, i, 0)),
        out_shape=jax.ShapeDtypeStruct((NC, NP, FH), jnp.float32),
    )(state, Wg, degp)


def _tc_node_mlp_in(accp, xsp, degp, state, bg, W1t, W1b):
    BLK = 1000

    def body(ac_ref, xs_ref, dg_ref, st_ref, bg_ref, w1t_ref, w1b_ref,
             o1_ref, o2_ref):
        acc = jnp.concatenate([ac_ref[0], ac_ref[1]], axis=1)
        xs = jnp.concatenate([xs_ref[0], xs_ref[1]], axis=1)
        deg = dg_ref[0][:, 0:1] + dg_ref[1][:, 0:1] + 1.0
        dinv = lax.rsqrt(deg)
        gcn = (acc + xs) * dinv + bg_ref[...]
        x = jnp.maximum(gcn, 0.0) + st_ref[...]
        o1_ref[...] = jnp.dot(x, w1t_ref[...], preferred_element_type=jnp.float32)
        o2_ref[...] = jnp.dot(x, w1b_ref[...], preferred_element_type=jnp.float32)

    return pl.pallas_call(
        body,
        grid=(N // BLK,),
        in_specs=[pl.BlockSpec((NC, BLK, FH), lambda i: (0, i, 0)),
                  pl.BlockSpec((NC, BLK, FH), lambda i: (0, i, 0)),
                  pl.BlockSpec((NC, BLK, FH), lambda i: (0, i, 0)),
                  pl.BlockSpec((BLK, F), lambda i: (i, 0)),
                  pl.BlockSpec((F,), lambda i: (0,)),
                  pl.BlockSpec((F, F), lambda i: (0, 0)),
                  pl.BlockSpec((F, F), lambda i: (0, 0))],
        out_specs=[pl.BlockSpec((BLK, F), lambda i: (i, 0)),
                   pl.BlockSpec((BLK, F), lambda i: (i, 0))],
        out_shape=[jax.ShapeDtypeStruct((N, F), jnp.float32),
                   jax.ShapeDtypeStruct((N, F), jnp.float32)],
    )(accp, xsp, degp, state, bg, W1t, W1b)


def _tc_head(g2, b1, W2, b2, Wmu, bmu):
    BLK = 3200

    def body(g_ref, b1_ref, w2_ref, b2_ref, wmu_ref, bmu_ref, o_ref):
        h = g_ref[0] + g_ref[1] + b1_ref[...]
        h = jnp.where(h > 0, h, 0.01 * h)
        h2 = jnp.dot(h, w2_ref[...], preferred_element_type=jnp.float32) + b2_ref[...]
        h2 = jnp.where(h2 > 0, h2, 0.01 * h2)
        m = jnp.dot(h2, wmu_ref[...], preferred_element_type=jnp.float32) + bmu_ref[...]
        mu = jax.nn.softplus(m)
        act = (jnp.tanh(mu) + 1.0) * (0.5 * (HIGH - LOW)) + LOW
        o_ref[...] = jnp.clip(act, LOW, HIGH)

    return pl.pallas_call(
        body,
        grid=(R // BLK,),
        in_specs=[pl.BlockSpec((2, BLK, F), lambda i: (0, i, 0)),
                  pl.BlockSpec((F,), lambda i: (0,)),
                  pl.BlockSpec((F, F), lambda i: (0, 0)),
                  pl.BlockSpec((F,), lambda i: (0,)),
                  pl.BlockSpec((F, 1), lambda i: (0, 0)),
                  pl.BlockSpec((1,), lambda i: (0,))],
        out_specs=pl.BlockSpec((BLK, 1), lambda i: (i, 0)),
        out_shape=jax.ShapeDtypeStruct((R, 1), jnp.float32),
    )(g2, b1, W2, b2, Wmu, bmu)


def kernel(state, edge_index, edges, deterministic,
           Wg, bg, W1, b1, W2, b2, Wmu, bmu, Wsig, bsig):
    del deterministic, Wsig, bsig
    pad = E2 - E
    src2 = jnp.concatenate([edge_index[0], jnp.zeros((pad,), jnp.int32)])
    dst2 = jnp.concatenate([edge_index[1], jnp.full((pad,), N, jnp.int32)])
    onesH = jnp.ones((CH, FH), jnp.float32)
    zerosH = jnp.zeros((ROWS_PER_TILE, FH), jnp.float32)

    degp = _sc_deg(dst2, onesH, zerosH)
    xsp = _tc_scale_split(state, Wg, degp)
    accp = _sc_gcn_agg(src2, dst2, xsp, zerosH)
    xs2, xd2 = _tc_node_mlp_in(accp, xsp, degp, state, bg,
                               W1[:F, :], W1[F:, :])

    boff = (jnp.arange(NB, dtype=jnp.int32) * A)[:, None]
    rpad = jnp.zeros((RP - R,), jnp.int32)
    i0 = jnp.concatenate([(boff + edges[:, 0][None, :]).reshape(-1), rpad])
    i1 = jnp.concatenate([(boff + edges[:, 1][None, :]).reshape(-1), rpad])
    g2 = _sc_pair_gather(xs2, xd2, i0, i1)

    act = _tc_head(g2, b1, W2, b2, Wmu, bmu)
    return act.reshape(NB, P)


# speedup vs baseline: 1.5181x; 1.3400x over previous
import functools

import jax
import jax.numpy as jnp
from jax import lax
from jax.experimental import pallas as pl
from jax.experimental.pallas import tpu as pltpu
from jax.experimental.pallas import tpu_sc as plsc

N = 10000
NP = 10240
F = 256
FH = 128
E = 160000
P = 8000
NB = 10
A = 1000
R = NB * P
LOW, HIGH = 0.0, 480.0

NC, NS = 2, 16
NW = NC * NS
CH = 128
CP = 128
E2 = 163840
EC2 = E2 // CH
ECT_B = EC2 // NS
RP = 81920
RC2 = RP // CP
RCT = RC2 // NW
ROWS_PER_TILE = NP // NS

_mesh = plsc.VectorSubcoreMesh(core_axis_name="c", subcore_axis_name="s")


@functools.partial(
    pl.kernel, mesh=_mesh,
    out_type=jax.ShapeDtypeStruct((NC, NP, FH), jnp.float32),
    scratch_types=[pltpu.VMEM((CH,), jnp.int32),
                   pltpu.VMEM((CH, FH), jnp.float32),
                   pltpu.VMEM_SHARED((NP, FH), jnp.float32),
                   pltpu.SemaphoreType.DMA],
)
def _sc_deg(dst2_hbm, ones_hbm, zeros_hbm, out_hbm, idx_v, ones_v, acc_sh, sem):
    c = lax.axis_index("c")
    s = lax.axis_index("s")
    wid = s * NC + c
    pltpu.sync_copy(ones_hbm, ones_v)
    sl = pl.ds(s * ROWS_PER_TILE, ROWS_PER_TILE)
    pltpu.sync_copy(zeros_hbm, acc_sh.at[sl])
    plsc.subcore_barrier()

    @pl.loop(wid, EC2, step=NW)
    def _(j):
        pltpu.sync_copy(dst2_hbm.at[pl.ds(j * CH, CH)], idx_v)
        pltpu.sync_copy(ones_v, acc_sh.at[idx_v], add=True)

    plsc.subcore_barrier()
    pltpu.sync_copy(acc_sh.at[sl], out_hbm.at[c].at[sl])


@functools.partial(
    pl.kernel, mesh=_mesh,
    out_type=jax.ShapeDtypeStruct((NC, NP, FH), jnp.float32),
    scratch_types=[pltpu.VMEM((CH,), jnp.int32),
                   pltpu.VMEM((CH,), jnp.int32),
                   pltpu.VMEM((CH,), jnp.int32),
                   pltpu.VMEM((CH,), jnp.int32),
                   pltpu.VMEM((CH, FH), jnp.float32),
                   pltpu.VMEM((CH, FH), jnp.float32),
                   pltpu.VMEM_SHARED((NP, FH), jnp.float32),
                   pltpu.SemaphoreType.DMA,
                   pltpu.SemaphoreType.DMA],
)
def _sc_gcn_agg(src2_hbm, dst2_hbm, xsp_hbm, zeros_hbm, out_hbm,
                si0, di0, si1, di1, rb0, rb1, acc_sh, g0, g1):
    c = lax.axis_index("c")
    s = lax.axis_index("s")
    sl = pl.ds(s * ROWS_PER_TILE, ROWS_PER_TILE)
    pltpu.sync_copy(zeros_hbm, acc_sh.at[sl])
    plsc.subcore_barrier()
    tbl = xsp_hbm.at[c]

    @pl.loop(s, EC2, step=NS)
    def _(j):
        rows = pl.ds(j * CH, CH)
        pltpu.sync_copy(src2_hbm.at[rows], si0)
        pltpu.sync_copy(dst2_hbm.at[rows], di0)
        pltpu.async_copy(tbl.at[si0], rb0, g0).wait()
        pltpu.sync_copy(rb0, acc_sh.at[di0], add=True)

    plsc.subcore_barrier()
    pltpu.sync_copy(acc_sh.at[sl], out_hbm.at[c].at[sl])


@functools.partial(
    pl.kernel, mesh=_mesh,
    out_type=jax.ShapeDtypeStruct((2, RP, F), jnp.float32),
    scratch_types=[pltpu.VMEM((CP,), jnp.int32),
                   pltpu.VMEM((CP,), jnp.int32),
                   pltpu.VMEM((CP, F), jnp.float32),
                   pltpu.VMEM((CP, F), jnp.float32),
                   pltpu.SemaphoreType.DMA,
                   pltpu.SemaphoreType.DMA],
)
def _sc_pair_gather(xs2_hbm, xd2_hbm, i02_hbm, i12_hbm, out_hbm,
                    i0v, i1v, ra, rb, g0, g1):
    c = lax.axis_index("c")
    s = lax.axis_index("s")
    wid = s * NC + c

    @pl.loop(wid, RC2, step=NW)
    def _(j):
        rows = pl.ds(j * CP, CP)
        pltpu.sync_copy(i02_hbm.at[rows], i0v)
        pltpu.sync_copy(i12_hbm.at[rows], i1v)
        cp0 = pltpu.async_copy(xs2_hbm.at[i0v], ra, g0)
        cp1 = pltpu.async_copy(xd2_hbm.at[i1v], rb, g1)
        cp0.wait()
        cp1.wait()
        pltpu.sync_copy(ra, out_hbm.at[0].at[rows])
        pltpu.sync_copy(rb, out_hbm.at[1].at[rows])


def _tc_scale_split(state, Wg, degp):
    BLK = 1000

    def body(st_ref, wg_ref, dg_ref, out_ref):
        xw = jnp.dot(st_ref[...], wg_ref[...], preferred_element_type=jnp.float32)
        deg = dg_ref[0][:, 0:1] + dg_ref[1][:, 0:1] + 1.0
        dinv = lax.rsqrt(deg)
        xs = xw * dinv
        out_ref[0] = xs[:, :FH]
        out_ref[1] = xs[:, FH:]

    return pl.pallas_call(
        body,
        grid=(N // BLK,),
        in_specs=[pl.BlockSpec((BLK, F), lambda i: (i, 0)),
                  pl.BlockSpec((F, F), lambda i: (0, 0)),
                  pl.BlockSpec((NC, BLK, FH), lambda i: (0, i, 0))],
        out_specs=pl.BlockSpec((NC, BLK, FH), lambda i: (0, i, 0)),
        out_shape=jax.ShapeDtypeStruct((NC, NP, FH), jnp.float32),
    )(state, Wg, degp)


def _tc_node_mlp_in(accp, xsp, degp, state, bg, W1t, W1b):
    BLK = 1000

    def body(ac_ref, xs_ref, dg_ref, st_ref, bg_ref, w1t_ref, w1b_ref,
             o1_ref, o2_ref):
        acc = jnp.concatenate([ac_ref[0], ac_ref[1]], axis=1)
        xs = jnp.concatenate([xs_ref[0], xs_ref[1]], axis=1)
        deg = dg_ref[0][:, 0:1] + dg_ref[1][:, 0:1] + 1.0
        dinv = lax.rsqrt(deg)
        gcn = (acc + xs) * dinv + bg_ref[...]
        x = jnp.maximum(gcn, 0.0) + st_ref[...]
        o1_ref[...] = jnp.dot(x, w1t_ref[...], preferred_element_type=jnp.float32)
        o2_ref[...] = jnp.dot(x, w1b_ref[...], preferred_element_type=jnp.float32)

    return pl.pallas_call(
        body,
        grid=(N // BLK,),
        in_specs=[pl.BlockSpec((NC, BLK, FH), lambda i: (0, i, 0)),
                  pl.BlockSpec((NC, BLK, FH), lambda i: (0, i, 0)),
                  pl.BlockSpec((NC, BLK, FH), lambda i: (0, i, 0)),
                  pl.BlockSpec((BLK, F), lambda i: (i, 0)),
                  pl.BlockSpec((F,), lambda i: (0,)),
                  pl.BlockSpec((F, F), lambda i: (0, 0)),
                  pl.BlockSpec((F, F), lambda i: (0, 0))],
        out_specs=[pl.BlockSpec((BLK, F), lambda i: (i, 0)),
                   pl.BlockSpec((BLK, F), lambda i: (i, 0))],
        out_shape=[jax.ShapeDtypeStruct((N, F), jnp.float32),
                   jax.ShapeDtypeStruct((N, F), jnp.float32)],
    )(accp, xsp, degp, state, bg, W1t, W1b)


def _tc_head(g2, b1, W2, b2, Wmu, bmu):
    BLK = 3200

    def body(g_ref, b1_ref, w2_ref, b2_ref, wmu_ref, bmu_ref, o_ref):
        h = g_ref[0] + g_ref[1] + b1_ref[...]
        h = jnp.where(h > 0, h, 0.01 * h)
        h2 = jnp.dot(h, w2_ref[...], preferred_element_type=jnp.float32) + b2_ref[...]
        h2 = jnp.where(h2 > 0, h2, 0.01 * h2)
        m = jnp.dot(h2, wmu_ref[...], preferred_element_type=jnp.float32) + bmu_ref[...]
        mu = jax.nn.softplus(m)
        act = (jnp.tanh(mu) + 1.0) * (0.5 * (HIGH - LOW)) + LOW
        o_ref[...] = jnp.clip(act, LOW, HIGH)

    return pl.pallas_call(
        body,
        grid=(R // BLK,),
        in_specs=[pl.BlockSpec((2, BLK, F), lambda i: (0, i, 0)),
                  pl.BlockSpec((F,), lambda i: (0,)),
                  pl.BlockSpec((F, F), lambda i: (0, 0)),
                  pl.BlockSpec((F,), lambda i: (0,)),
                  pl.BlockSpec((F, 1), lambda i: (0, 0)),
                  pl.BlockSpec((1,), lambda i: (0,))],
        out_specs=pl.BlockSpec((BLK, 1), lambda i: (i, 0)),
        out_shape=jax.ShapeDtypeStruct((R, 1), jnp.float32),
    )(g2, b1, W2, b2, Wmu, bmu)


def kernel(state, edge_index, edges, deterministic,
           Wg, bg, W1, b1, W2, b2, Wmu, bmu, Wsig, bsig):
    del deterministic, Wsig, bsig
    pad = E2 - E
    spread = jnp.arange(pad, dtype=jnp.int32)
    src2 = jnp.concatenate([edge_index[0], spread % N])
    dst2 = jnp.concatenate([edge_index[1], N + (spread % (NP - N))])
    onesH = jnp.ones((CH, FH), jnp.float32)
    zerosH = jnp.zeros((ROWS_PER_TILE, FH), jnp.float32)

    degp = _sc_deg(dst2, onesH, zerosH)
    xsp = _tc_scale_split(state, Wg, degp)
    accp = _sc_gcn_agg(src2, dst2, xsp, zerosH)
    xs2, xd2 = _tc_node_mlp_in(accp, xsp, degp, state, bg,
                               W1[:F, :], W1[F:, :])

    boff = (jnp.arange(NB, dtype=jnp.int32) * A)[:, None]
    rpad = (jnp.arange(RP - R, dtype=jnp.int32) * 997) % N
    i0 = jnp.concatenate([(boff + edges[:, 0][None, :]).reshape(-1), rpad])
    i1 = jnp.concatenate([(boff + edges[:, 1][None, :]).reshape(-1), rpad])
    g2 = _sc_pair_gather(xs2, xd2, i0, i1)

    act = _tc_head(g2, b1, W2, b2, Wmu, bmu)
    return act.reshape(NB, P)


# speedup vs baseline: 1.5269x; 1.0058x over previous
import functools

import jax
import jax.numpy as jnp
from jax import lax
from jax.experimental import pallas as pl
from jax.experimental.pallas import tpu as pltpu
from jax.experimental.pallas import tpu_sc as plsc

N = 10000
NP = 10240
F = 256
FH = 128
E = 160000
P = 8000
NB = 10
A = 1000
R = NB * P
LOW, HIGH = 0.0, 480.0

NC, NS = 2, 16
NW = NC * NS
CH = 128
CP = 128
E2 = 163840
EC2 = E2 // CH
ECT_B = EC2 // NS
RH = R // 2
RP = 40960
RC2 = RP // CP
RCT = RC2 // NW
ROWS_PER_TILE = NP // NS

_mesh = plsc.VectorSubcoreMesh(core_axis_name="c", subcore_axis_name="s")


@functools.partial(
    pl.kernel, mesh=_mesh,
    out_type=jax.ShapeDtypeStruct((NC, NP, FH), jnp.float32),
    scratch_types=[pltpu.VMEM((CH,), jnp.int32),
                   pltpu.VMEM((CH, FH), jnp.float32),
                   pltpu.VMEM_SHARED((NP, FH), jnp.float32),
                   pltpu.SemaphoreType.DMA],
)
def _sc_deg(dst2_hbm, ones_hbm, zeros_hbm, out_hbm, idx_v, ones_v, acc_sh, sem):
    c = lax.axis_index("c")
    s = lax.axis_index("s")
    wid = s * NC + c
    pltpu.sync_copy(ones_hbm, ones_v)
    sl = pl.ds(s * ROWS_PER_TILE, ROWS_PER_TILE)
    pltpu.sync_copy(zeros_hbm, acc_sh.at[sl])
    plsc.subcore_barrier()

    @pl.loop(wid, EC2, step=NW)
    def _(j):
        pltpu.sync_copy(dst2_hbm.at[pl.ds(j * CH, CH)], idx_v)
        pltpu.sync_copy(ones_v, acc_sh.at[idx_v], add=True)

    plsc.subcore_barrier()
    pltpu.sync_copy(acc_sh.at[sl], out_hbm.at[c].at[sl])


@functools.partial(
    pl.kernel, mesh=_mesh,
    out_type=jax.ShapeDtypeStruct((NC, NP, FH), jnp.float32),
    scratch_types=[pltpu.VMEM((CH,), jnp.int32),
                   pltpu.VMEM((CH,), jnp.int32),
                   pltpu.VMEM((CH,), jnp.int32),
                   pltpu.VMEM((CH,), jnp.int32),
                   pltpu.VMEM((CH, FH), jnp.float32),
                   pltpu.VMEM((CH, FH), jnp.float32),
                   pltpu.VMEM_SHARED((NP, FH), jnp.float32),
                   pltpu.SemaphoreType.DMA,
                   pltpu.SemaphoreType.DMA],
)
def _sc_gcn_agg(src2_hbm, dst2_hbm, xsp_hbm, zeros_hbm, out_hbm,
                si0, di0, si1, di1, rb0, rb1, acc_sh, g0, g1):
    c = lax.axis_index("c")
    s = lax.axis_index("s")
    sl = pl.ds(s * ROWS_PER_TILE, ROWS_PER_TILE)
    pltpu.sync_copy(zeros_hbm, acc_sh.at[sl])
    plsc.subcore_barrier()
    tbl = xsp_hbm.at[c]

    @pl.loop(s, EC2, step=NS)
    def _(j):
        rows = pl.ds(j * CH, CH)
        pltpu.sync_copy(src2_hbm.at[rows], si0)
        pltpu.sync_copy(dst2_hbm.at[rows], di0)
        pltpu.async_copy(tbl.at[si0], rb0, g0).wait()
        pltpu.sync_copy(rb0, acc_sh.at[di0], add=True)

    plsc.subcore_barrier()
    pltpu.sync_copy(acc_sh.at[sl], out_hbm.at[c].at[sl])


@functools.partial(
    pl.kernel, mesh=_mesh,
    out_type=jax.ShapeDtypeStruct((2, RP, F), jnp.float32),
    scratch_types=[pltpu.VMEM((CP,), jnp.int32),
                   pltpu.VMEM((CP,), jnp.int32),
                   pltpu.VMEM((CP, F), jnp.float32),
                   pltpu.VMEM((CP, F), jnp.float32),
                   pltpu.SemaphoreType.DMA,
                   pltpu.SemaphoreType.DMA],
)
def _sc_pair_gather(xs2_hbm, xd2_hbm, i02_hbm, i12_hbm, out_hbm,
                    i0v, i1v, ra, rb, g0, g1):
    c = lax.axis_index("c")
    s = lax.axis_index("s")
    wid = s * NC + c

    @pl.loop(wid, RC2, step=NW)
    def _(j):
        rows = pl.ds(j * CP, CP)
        pltpu.sync_copy(i02_hbm.at[rows], i0v)
        pltpu.sync_copy(i12_hbm.at[rows], i1v)
        cp0 = pltpu.async_copy(xs2_hbm.at[i0v], ra, g0)
        cp1 = pltpu.async_copy(xd2_hbm.at[i1v], rb, g1)
        cp0.wait()
        cp1.wait()
        pltpu.sync_copy(ra, out_hbm.at[0].at[rows])
        pltpu.sync_copy(rb, out_hbm.at[1].at[rows])


def _tc_scale_split(state, Wg, degp):
    BLK = 1000

    def body(st_ref, wg_ref, dg_ref, out_ref):
        xw = jnp.dot(st_ref[...], wg_ref[...], preferred_element_type=jnp.float32)
        deg = dg_ref[0][:, 0:1] + dg_ref[1][:, 0:1] + 1.0
        dinv = lax.rsqrt(deg)
        xs = xw * dinv
        out_ref[0] = xs[:, :FH]
        out_ref[1] = xs[:, FH:]

    return pl.pallas_call(
        body,
        grid=(N // BLK,),
        in_specs=[pl.BlockSpec((BLK, F), lambda i: (i, 0)),
                  pl.BlockSpec((F, F), lambda i: (0, 0)),
                  pl.BlockSpec((NC, BLK, FH), lambda i: (0, i, 0))],
        out_specs=pl.BlockSpec((NC, BLK, FH), lambda i: (0, i, 0)),
        out_shape=jax.ShapeDtypeStruct((NC, NP, FH), jnp.float32),
    )(state, Wg, degp)


def _tc_node_mlp_in(accp, xsp, degp, state, bg, W1t, W1b):
    BLK = 1000

    def body(ac_ref, xs_ref, dg_ref, st_ref, bg_ref, w1t_ref, w1b_ref,
             o1_ref, o2_ref):
        acc = jnp.concatenate([ac_ref[0], ac_ref[1]], axis=1)
        xs = jnp.concatenate([xs_ref[0], xs_ref[1]], axis=1)
        deg = dg_ref[0][:, 0:1] + dg_ref[1][:, 0:1] + 1.0
        dinv = lax.rsqrt(deg)
        gcn = (acc + xs) * dinv + bg_ref[...]
        x = jnp.maximum(gcn, 0.0) + st_ref[...]
        o1_ref[...] = jnp.dot(x, w1t_ref[...], preferred_element_type=jnp.float32)
        o2_ref[...] = jnp.dot(x, w1b_ref[...], preferred_element_type=jnp.float32)

    return pl.pallas_call(
        body,
        grid=(N // BLK,),
        in_specs=[pl.BlockSpec((NC, BLK, FH), lambda i: (0, i, 0)),
                  pl.BlockSpec((NC, BLK, FH), lambda i: (0, i, 0)),
                  pl.BlockSpec((NC, BLK, FH), lambda i: (0, i, 0)),
                  pl.BlockSpec((BLK, F), lambda i: (i, 0)),
                  pl.BlockSpec((F,), lambda i: (0,)),
                  pl.BlockSpec((F, F), lambda i: (0, 0)),
                  pl.BlockSpec((F, F), lambda i: (0, 0))],
        out_specs=[pl.BlockSpec((BLK, F), lambda i: (i, 0)),
                   pl.BlockSpec((BLK, F), lambda i: (i, 0))],
        out_shape=[jax.ShapeDtypeStruct((N, F), jnp.float32),
                   jax.ShapeDtypeStruct((N, F), jnp.float32)],
    )(accp, xsp, degp, state, bg, W1t, W1b)


def _tc_head(g2, b1, W2, b2, Wmu, bmu):
    BLK = 2000

    def body(g_ref, b1_ref, w2_ref, b2_ref, wmu_ref, bmu_ref, o_ref):
        h = g_ref[0] + g_ref[1] + b1_ref[...]
        h = jnp.where(h > 0, h, 0.01 * h)
        h2 = jnp.dot(h, w2_ref[...], preferred_element_type=jnp.float32) + b2_ref[...]
        h2 = jnp.where(h2 > 0, h2, 0.01 * h2)
        m = jnp.dot(h2, wmu_ref[...], preferred_element_type=jnp.float32) + bmu_ref[...]
        mu = jax.nn.softplus(m)
        act = (jnp.tanh(mu) + 1.0) * (0.5 * (HIGH - LOW)) + LOW
        o_ref[...] = jnp.clip(act, LOW, HIGH)

    return pl.pallas_call(
        body,
        grid=(RH // BLK,),
        in_specs=[pl.BlockSpec((2, BLK, F), lambda i: (0, i, 0)),
                  pl.BlockSpec((F,), lambda i: (0,)),
                  pl.BlockSpec((F, F), lambda i: (0, 0)),
                  pl.BlockSpec((F,), lambda i: (0,)),
                  pl.BlockSpec((F, 1), lambda i: (0, 0)),
                  pl.BlockSpec((1,), lambda i: (0,))],
        out_specs=pl.BlockSpec((BLK, 1), lambda i: (i, 0)),
        out_shape=jax.ShapeDtypeStruct((RH, 1), jnp.float32),
    )(g2, b1, W2, b2, Wmu, bmu)


def kernel(state, edge_index, edges, deterministic,
           Wg, bg, W1, b1, W2, b2, Wmu, bmu, Wsig, bsig):
    del deterministic, Wsig, bsig
    pad = E2 - E
    spread = jnp.arange(pad, dtype=jnp.int32)
    src2 = jnp.concatenate([edge_index[0], spread % N])
    dst2 = jnp.concatenate([edge_index[1], N + (spread % (NP - N))])
    onesH = jnp.ones((CH, FH), jnp.float32)
    zerosH = jnp.zeros((ROWS_PER_TILE, FH), jnp.float32)

    degp = _sc_deg(dst2, onesH, zerosH)
    xsp = _tc_scale_split(state, Wg, degp)
    accp = _sc_gcn_agg(src2, dst2, xsp, zerosH)
    xs2, xd2 = _tc_node_mlp_in(accp, xsp, degp, state, bg,
                               W1[:F, :], W1[F:, :])

    boff = (jnp.arange(NB, dtype=jnp.int32) * A)[:, None]
    rpad = (jnp.arange(RP - RH, dtype=jnp.int32) * 997) % N
    i0 = (boff + edges[:, 0][None, :]).reshape(-1)
    i1 = (boff + edges[:, 1][None, :]).reshape(-1)

    acts = []
    for h in range(2):
        i0h = jnp.concatenate([lax.dynamic_slice(i0, (h * RH,), (RH,)), rpad])
        i1h = jnp.concatenate([lax.dynamic_slice(i1, (h * RH,), (RH,)), rpad])
        g2 = _sc_pair_gather(xs2, xd2, i0h, i1h)
        acts.append(_tc_head(g2, b1, W2, b2, Wmu, bmu))
    act = jnp.concatenate(acts, axis=0)
    return act.reshape(NB, P)


# speedup vs baseline: 1.6593x; 1.0868x over previous
import functools

import jax
import jax.numpy as jnp
from jax import lax
from jax.experimental import pallas as pl
from jax.experimental.pallas import tpu as pltpu
from jax.experimental.pallas import tpu_sc as plsc

N = 10000
NP = 10240
F = 256
FH = 128
E = 160000
P = 8000
NB = 10
A = 1000
R = NB * P
LOW, HIGH = 0.0, 480.0

NC, NS = 2, 16
NW = NC * NS
CH = 128
CP = 128
E2 = 163840
EC2 = E2 // CH
ECT_B = EC2 // NS
RH = R // 2
RP = 40960
RC2 = RP // CP
RCT = RC2 // NW
ROWS_PER_TILE = NP // NS

_mesh = plsc.VectorSubcoreMesh(core_axis_name="c", subcore_axis_name="s")


@functools.partial(
    pl.kernel, mesh=_mesh,
    out_type=jax.ShapeDtypeStruct((NC, NP, FH), jnp.float32),
    scratch_types=[pltpu.VMEM((CH,), jnp.int32),
                   pltpu.VMEM((CH, FH), jnp.float32),
                   pltpu.VMEM_SHARED((NP, FH), jnp.float32),
                   pltpu.SemaphoreType.DMA],
)
def _sc_deg(dst2_hbm, ones_hbm, zeros_hbm, out_hbm, idx_v, ones_v, acc_sh, sem):
    c = lax.axis_index("c")
    s = lax.axis_index("s")
    wid = s * NC + c
    pltpu.sync_copy(ones_hbm, ones_v)
    sl = pl.ds(s * ROWS_PER_TILE, ROWS_PER_TILE)
    pltpu.sync_copy(zeros_hbm, acc_sh.at[sl])
    plsc.subcore_barrier()

    @pl.loop(wid, EC2, step=NW)
    def _(j):
        pltpu.sync_copy(dst2_hbm.at[pl.ds(j * CH, CH)], idx_v)
        pltpu.sync_copy(ones_v, acc_sh.at[idx_v], add=True)

    plsc.subcore_barrier()
    pltpu.sync_copy(acc_sh.at[sl], out_hbm.at[c].at[sl])


@functools.partial(
    pl.kernel, mesh=_mesh,
    out_type=jax.ShapeDtypeStruct((NC, NP, FH), jnp.float32),
    scratch_types=[pltpu.VMEM((ECT_B * CH,), jnp.int32),
                   pltpu.VMEM((CH,), jnp.int32),
                   pltpu.VMEM((CH, FH), jnp.float32),
                   pltpu.VMEM_SHARED((NP, FH), jnp.float32),
                   pltpu.SemaphoreType.DMA],
)
def _sc_gcn_agg(src2_hbm, dst2_hbm, xsp_hbm, zeros_hbm, out_hbm,
                sidx_all, di0, rb0, acc_sh, g0):
    c = lax.axis_index("c")
    s = lax.axis_index("s")
    sl = pl.ds(s * ROWS_PER_TILE, ROWS_PER_TILE)
    pltpu.sync_copy(zeros_hbm, acc_sh.at[sl])
    pltpu.sync_copy(src2_hbm.at[pl.ds(s * ECT_B * CH, ECT_B * CH)], sidx_all)
    plsc.subcore_barrier()
    tbl = xsp_hbm.at[c]

    @pl.loop(0, ECT_B)
    def _(k):
        j = s * ECT_B + k
        pltpu.sync_copy(dst2_hbm.at[pl.ds(j * CH, CH)], di0)
        pltpu.async_copy(tbl.at[sidx_all.at[pl.ds(k * CH, CH)]], rb0, g0).wait()
        pltpu.sync_copy(rb0, acc_sh.at[di0], add=True)

    plsc.subcore_barrier()
    pltpu.sync_copy(acc_sh.at[sl], out_hbm.at[c].at[sl])


@functools.partial(
    pl.kernel, mesh=_mesh,
    out_type=jax.ShapeDtypeStruct((2, RP, F), jnp.float32),
    scratch_types=[pltpu.VMEM((RCT * CP,), jnp.int32),
                   pltpu.VMEM((RCT * CP,), jnp.int32),
                   pltpu.VMEM((CP, F), jnp.float32),
                   pltpu.VMEM((CP, F), jnp.float32),
                   pltpu.SemaphoreType.DMA,
                   pltpu.SemaphoreType.DMA],
)
def _sc_pair_gather(xs2_hbm, xd2_hbm, i02_hbm, i12_hbm, out_hbm,
                    i0v, i1v, ra, rb, g0, g1):
    c = lax.axis_index("c")
    s = lax.axis_index("s")
    wid = s * NC + c
    base = wid * RCT * CP
    pltpu.sync_copy(i02_hbm.at[pl.ds(base, RCT * CP)], i0v)
    pltpu.sync_copy(i12_hbm.at[pl.ds(base, RCT * CP)], i1v)

    @pl.loop(0, RCT)
    def _(k):
        rows = pl.ds(base + k * CP, CP)
        isl = pl.ds(k * CP, CP)
        cp0 = pltpu.async_copy(xs2_hbm.at[i0v.at[isl]], ra, g0)
        cp1 = pltpu.async_copy(xd2_hbm.at[i1v.at[isl]], rb, g1)
        cp0.wait()
        cp1.wait()
        pltpu.sync_copy(ra, out_hbm.at[0].at[rows])
        pltpu.sync_copy(rb, out_hbm.at[1].at[rows])


def _tc_scale_split(state, Wg, degp):
    BLK = 1000

    def body(st_ref, wg_ref, dg_ref, out_ref):
        xw = jnp.dot(st_ref[...], wg_ref[...], preferred_element_type=jnp.float32)
        deg = dg_ref[0][:, 0:1] + dg_ref[1][:, 0:1] + 1.0
        dinv = lax.rsqrt(deg)
        xs = xw * dinv
        out_ref[0] = xs[:, :FH]
        out_ref[1] = xs[:, FH:]

    return pl.pallas_call(
        body,
        grid=(N // BLK,),
        in_specs=[pl.BlockSpec((BLK, F), lambda i: (i, 0)),
                  pl.BlockSpec((F, F), lambda i: (0, 0)),
                  pl.BlockSpec((NC, BLK, FH), lambda i: (0, i, 0))],
        out_specs=pl.BlockSpec((NC, BLK, FH), lambda i: (0, i, 0)),
        out_shape=jax.ShapeDtypeStruct((NC, NP, FH), jnp.float32),
    )(state, Wg, degp)


def _tc_node_mlp_in(accp, xsp, degp, state, bg, W1t, W1b):
    BLK = 1000

    def body(ac_ref, xs_ref, dg_ref, st_ref, bg_ref, w1t_ref, w1b_ref,
             o1_ref, o2_ref):
        acc = jnp.concatenate([ac_ref[0], ac_ref[1]], axis=1)
        xs = jnp.concatenate([xs_ref[0], xs_ref[1]], axis=1)
        deg = dg_ref[0][:, 0:1] + dg_ref[1][:, 0:1] + 1.0
        dinv = lax.rsqrt(deg)
        gcn = (acc + xs) * dinv + bg_ref[...]
        x = jnp.maximum(gcn, 0.0) + st_ref[...]
        o1_ref[...] = jnp.dot(x, w1t_ref[...], preferred_element_type=jnp.float32)
        o2_ref[...] = jnp.dot(x, w1b_ref[...], preferred_element_type=jnp.float32)

    return pl.pallas_call(
        body,
        grid=(N // BLK,),
        in_specs=[pl.BlockSpec((NC, BLK, FH), lambda i: (0, i, 0)),
                  pl.BlockSpec((NC, BLK, FH), lambda i: (0, i, 0)),
                  pl.BlockSpec((NC, BLK, FH), lambda i: (0, i, 0)),
                  pl.BlockSpec((BLK, F), lambda i: (i, 0)),
                  pl.BlockSpec((F,), lambda i: (0,)),
                  pl.BlockSpec((F, F), lambda i: (0, 0)),
                  pl.BlockSpec((F, F), lambda i: (0, 0))],
        out_specs=[pl.BlockSpec((BLK, F), lambda i: (i, 0)),
                   pl.BlockSpec((BLK, F), lambda i: (i, 0))],
        out_shape=[jax.ShapeDtypeStruct((N, F), jnp.float32),
                   jax.ShapeDtypeStruct((N, F), jnp.float32)],
    )(accp, xsp, degp, state, bg, W1t, W1b)


def _tc_head(g2, b1, W2, b2, Wmu, bmu):
    BLK = 2000

    def body(g_ref, b1_ref, w2_ref, b2_ref, wmu_ref, bmu_ref, o_ref):
        h = g_ref[0] + g_ref[1] + b1_ref[...]
        h = jnp.where(h > 0, h, 0.01 * h)
        h2 = jnp.dot(h, w2_ref[...], preferred_element_type=jnp.float32) + b2_ref[...]
        h2 = jnp.where(h2 > 0, h2, 0.01 * h2)
        m = jnp.dot(h2, wmu_ref[...], preferred_element_type=jnp.float32) + bmu_ref[...]
        mu = jax.nn.softplus(m)
        act = (jnp.tanh(mu) + 1.0) * (0.5 * (HIGH - LOW)) + LOW
        o_ref[...] = jnp.clip(act, LOW, HIGH)

    return pl.pallas_call(
        body,
        grid=(RH // BLK,),
        in_specs=[pl.BlockSpec((2, BLK, F), lambda i: (0, i, 0)),
                  pl.BlockSpec((F,), lambda i: (0,)),
                  pl.BlockSpec((F, F), lambda i: (0, 0)),
                  pl.BlockSpec((F,), lambda i: (0,)),
                  pl.BlockSpec((F, 1), lambda i: (0, 0)),
                  pl.BlockSpec((1,), lambda i: (0,))],
        out_specs=pl.BlockSpec((BLK, 1), lambda i: (i, 0)),
        out_shape=jax.ShapeDtypeStruct((RH, 1), jnp.float32),
    )(g2, b1, W2, b2, Wmu, bmu)


def kernel(state, edge_index, edges, deterministic,
           Wg, bg, W1, b1, W2, b2, Wmu, bmu, Wsig, bsig):
    del deterministic, Wsig, bsig
    pad = E2 - E
    spread = jnp.arange(pad, dtype=jnp.int32)
    src2 = jnp.concatenate([edge_index[0], spread % N])
    dst2 = jnp.concatenate([edge_index[1], N + (spread % (NP - N))])
    onesH = jnp.ones((CH, FH), jnp.float32)
    zerosH = jnp.zeros((ROWS_PER_TILE, FH), jnp.float32)

    degp = _sc_deg(dst2, onesH, zerosH)
    xsp = _tc_scale_split(state, Wg, degp)
    accp = _sc_gcn_agg(src2, dst2, xsp, zerosH)
    xs2, xd2 = _tc_node_mlp_in(accp, xsp, degp, state, bg,
                               W1[:F, :], W1[F:, :])

    boff = (jnp.arange(NB, dtype=jnp.int32) * A)[:, None]
    rpad = (jnp.arange(RP - RH, dtype=jnp.int32) * 997) % N
    i0 = (boff + edges[:, 0][None, :]).reshape(-1)
    i1 = (boff + edges[:, 1][None, :]).reshape(-1)

    acts = []
    for h in range(2):
        i0h = jnp.concatenate([lax.dynamic_slice(i0, (h * RH,), (RH,)), rpad])
        i1h = jnp.concatenate([lax.dynamic_slice(i1, (h * RH,), (RH,)), rpad])
        g2 = _sc_pair_gather(xs2, xd2, i0h, i1h)
        acts.append(_tc_head(g2, b1, W2, b2, Wmu, bmu))
    act = jnp.concatenate(acts, axis=0)
    return act.reshape(NB, P)


# speedup vs baseline: 1.7753x; 1.0699x over previous
import functools

import jax
import jax.numpy as jnp
from jax import lax
from jax.experimental import pallas as pl
from jax.experimental.pallas import tpu as pltpu
from jax.experimental.pallas import tpu_sc as plsc

N = 10000
NP = 10240
F = 256
FH = 128
E = 160000
P = 8000
NB = 10
A = 1000
R = NB * P
LOW, HIGH = 0.0, 480.0

NC, NS = 2, 16
NW = NC * NS
CH = 128
CP = 128
E2 = 163840
EC2 = E2 // CH
ECT_B = EC2 // NS
RH = R // 2
RP = 40960
RC2 = RP // CP
RCT = RC2 // NW
ROWS_PER_TILE = NP // NS

_mesh = plsc.VectorSubcoreMesh(core_axis_name="c", subcore_axis_name="s")


@functools.partial(
    pl.kernel, mesh=_mesh,
    out_type=jax.ShapeDtypeStruct((NC, NP, FH), jnp.float32),
    scratch_types=[pltpu.VMEM((CH,), jnp.int32),
                   pltpu.VMEM((CH, FH), jnp.float32),
                   pltpu.VMEM_SHARED((NP, FH), jnp.float32),
                   pltpu.SemaphoreType.DMA],
)
def _sc_deg(dst2_hbm, ones_hbm, zeros_hbm, out_hbm, idx_v, ones_v, acc_sh, sem):
    c = lax.axis_index("c")
    s = lax.axis_index("s")
    wid = s * NC + c
    pltpu.sync_copy(ones_hbm, ones_v)
    sl = pl.ds(s * ROWS_PER_TILE, ROWS_PER_TILE)
    pltpu.sync_copy(zeros_hbm, acc_sh.at[sl])
    plsc.subcore_barrier()

    @pl.loop(wid, EC2, step=NW)
    def _(j):
        pltpu.sync_copy(dst2_hbm.at[pl.ds(j * CH, CH)], idx_v)
        pltpu.sync_copy(ones_v, acc_sh.at[idx_v], add=True)

    plsc.subcore_barrier()
    pltpu.sync_copy(acc_sh.at[sl], out_hbm.at[c].at[sl])


@functools.partial(
    pl.kernel, mesh=_mesh,
    out_type=jax.ShapeDtypeStruct((NC, NP, FH), jnp.float32),
    scratch_types=[pltpu.VMEM((ECT_B * CH,), jnp.int32),
                   pltpu.VMEM((ECT_B, CH), jnp.int32),
                   pltpu.VMEM((CH, FH), jnp.float32),
                   pltpu.VMEM_SHARED((NP, FH), jnp.float32),
                   pltpu.SemaphoreType.DMA],
)
def _sc_gcn_agg(src2_hbm, dst2_hbm, xsp_hbm, zeros_hbm, out_hbm,
                sidx_all, didx_all, rb0, acc_sh, g0):
    c = lax.axis_index("c")
    s = lax.axis_index("s")
    sl = pl.ds(s * ROWS_PER_TILE, ROWS_PER_TILE)
    pltpu.sync_copy(zeros_hbm, acc_sh.at[sl])
    pltpu.sync_copy(src2_hbm.at[pl.ds(s * ECT_B * CH, ECT_B * CH)], sidx_all)
    pltpu.sync_copy(dst2_hbm.at[pl.ds(s * ECT_B, ECT_B)], didx_all)
    plsc.subcore_barrier()
    tbl = xsp_hbm.at[c]

    @pl.loop(0, ECT_B)
    def _(k):
        pltpu.async_copy(tbl.at[sidx_all.at[pl.ds(k * CH, CH)]], rb0, g0).wait()
        pltpu.sync_copy(rb0, acc_sh.at[didx_all.at[k]], add=True)

    plsc.subcore_barrier()
    pltpu.sync_copy(acc_sh.at[sl], out_hbm.at[c].at[sl])


@functools.partial(
    pl.kernel, mesh=_mesh,
    out_type=jax.ShapeDtypeStruct((2, RP, F), jnp.float32),
    scratch_types=[pltpu.VMEM((RCT * CP,), jnp.int32),
                   pltpu.VMEM((RCT * CP,), jnp.int32),
                   pltpu.VMEM((CP, F), jnp.float32),
                   pltpu.VMEM((CP, F), jnp.float32),
                   pltpu.SemaphoreType.DMA,
                   pltpu.SemaphoreType.DMA],
)
def _sc_pair_gather(xs2_hbm, xd2_hbm, i02_hbm, i12_hbm, out_hbm,
                    i0v, i1v, ra, rb, g0, g1):
    c = lax.axis_index("c")
    s = lax.axis_index("s")
    wid = s * NC + c
    base = wid * RCT * CP
    pltpu.sync_copy(i02_hbm.at[pl.ds(base, RCT * CP)], i0v)
    pltpu.sync_copy(i12_hbm.at[pl.ds(base, RCT * CP)], i1v)

    @pl.loop(0, RCT)
    def _(k):
        rows = pl.ds(base + k * CP, CP)
        isl = pl.ds(k * CP, CP)
        cp0 = pltpu.async_copy(xs2_hbm.at[i0v.at[isl]], ra, g0)
        cp1 = pltpu.async_copy(xd2_hbm.at[i1v.at[isl]], rb, g1)
        cp0.wait()
        cp1.wait()
        pltpu.sync_copy(ra, out_hbm.at[0].at[rows])
        pltpu.sync_copy(rb, out_hbm.at[1].at[rows])


def _tc_scale_split(state, Wg, degp):
    BLK = 1000

    def body(st_ref, wg_ref, dg_ref, out_ref):
        xw = jnp.dot(st_ref[...], wg_ref[...], preferred_element_type=jnp.float32)
        deg = dg_ref[0][:, 0:1] + dg_ref[1][:, 0:1] + 1.0
        dinv = lax.rsqrt(deg)
        xs = xw * dinv
        out_ref[0] = xs[:, :FH]
        out_ref[1] = xs[:, FH:]

    return pl.pallas_call(
        body,
        grid=(N // BLK,),
        in_specs=[pl.BlockSpec((BLK, F), lambda i: (i, 0)),
                  pl.BlockSpec((F, F), lambda i: (0, 0)),
                  pl.BlockSpec((NC, BLK, FH), lambda i: (0, i, 0))],
        out_specs=pl.BlockSpec((NC, BLK, FH), lambda i: (0, i, 0)),
        out_shape=jax.ShapeDtypeStruct((NC, NP, FH), jnp.float32),
    )(state, Wg, degp)


def _tc_node_mlp_in(accp, xsp, degp, state, bg, W1t, W1b):
    BLK = 1000

    def body(ac_ref, xs_ref, dg_ref, st_ref, bg_ref, w1t_ref, w1b_ref,
             o1_ref, o2_ref):
        acc = jnp.concatenate([ac_ref[0], ac_ref[1]], axis=1)
        xs = jnp.concatenate([xs_ref[0], xs_ref[1]], axis=1)
        deg = dg_ref[0][:, 0:1] + dg_ref[1][:, 0:1] + 1.0
        dinv = lax.rsqrt(deg)
        gcn = (acc + xs) * dinv + bg_ref[...]
        x = jnp.maximum(gcn, 0.0) + st_ref[...]
        o1_ref[...] = jnp.dot(x, w1t_ref[...], preferred_element_type=jnp.float32)
        o2_ref[...] = jnp.dot(x, w1b_ref[...], preferred_element_type=jnp.float32)

    return pl.pallas_call(
        body,
        grid=(N // BLK,),
        in_specs=[pl.BlockSpec((NC, BLK, FH), lambda i: (0, i, 0)),
                  pl.BlockSpec((NC, BLK, FH), lambda i: (0, i, 0)),
                  pl.BlockSpec((NC, BLK, FH), lambda i: (0, i, 0)),
                  pl.BlockSpec((BLK, F), lambda i: (i, 0)),
                  pl.BlockSpec((F,), lambda i: (0,)),
                  pl.BlockSpec((F, F), lambda i: (0, 0)),
                  pl.BlockSpec((F, F), lambda i: (0, 0))],
        out_specs=[pl.BlockSpec((BLK, F), lambda i: (i, 0)),
                   pl.BlockSpec((BLK, F), lambda i: (i, 0))],
        out_shape=[jax.ShapeDtypeStruct((N, F), jnp.float32),
                   jax.ShapeDtypeStruct((N, F), jnp.float32)],
    )(accp, xsp, degp, state, bg, W1t, W1b)


def _tc_head(g2, b1, W2, b2, Wmu, bmu):
    BLK = 2000

    def body(g_ref, b1_ref, w2_ref, b2_ref, wmu_ref, bmu_ref, o_ref):
        h = g_ref[0] + g_ref[1] + b1_ref[...]
        h = jnp.where(h > 0, h, 0.01 * h)
        h2 = jnp.dot(h, w2_ref[...], preferred_element_type=jnp.float32) + b2_ref[...]
        h2 = jnp.where(h2 > 0, h2, 0.01 * h2)
        m = jnp.dot(h2, wmu_ref[...], preferred_element_type=jnp.float32) + bmu_ref[...]
        mu = jax.nn.softplus(m)
        act = (jnp.tanh(mu) + 1.0) * (0.5 * (HIGH - LOW)) + LOW
        o_ref[...] = jnp.clip(act, LOW, HIGH)

    return pl.pallas_call(
        body,
        grid=(RH // BLK,),
        in_specs=[pl.BlockSpec((2, BLK, F), lambda i: (0, i, 0)),
                  pl.BlockSpec((F,), lambda i: (0,)),
                  pl.BlockSpec((F, F), lambda i: (0, 0)),
                  pl.BlockSpec((F,), lambda i: (0,)),
                  pl.BlockSpec((F, 1), lambda i: (0, 0)),
                  pl.BlockSpec((1,), lambda i: (0,))],
        out_specs=pl.BlockSpec((BLK, 1), lambda i: (i, 0)),
        out_shape=jax.ShapeDtypeStruct((RH, 1), jnp.float32),
    )(g2, b1, W2, b2, Wmu, bmu)


def kernel(state, edge_index, edges, deterministic,
           Wg, bg, W1, b1, W2, b2, Wmu, bmu, Wsig, bsig):
    del deterministic, Wsig, bsig
    pad = E2 - E
    spread = jnp.arange(pad, dtype=jnp.int32)
    src2 = jnp.concatenate([edge_index[0], spread % N])
    dst2 = jnp.concatenate([edge_index[1], N + (spread % (NP - N))])
    onesH = jnp.ones((CH, FH), jnp.float32)
    zerosH = jnp.zeros((ROWS_PER_TILE, FH), jnp.float32)

    degp = _sc_deg(dst2, onesH, zerosH)
    xsp = _tc_scale_split(state, Wg, degp)
    accp = _sc_gcn_agg(src2, dst2.reshape(EC2, CH), xsp, zerosH)
    xs2, xd2 = _tc_node_mlp_in(accp, xsp, degp, state, bg,
                               W1[:F, :], W1[F:, :])

    boff = (jnp.arange(NB, dtype=jnp.int32) * A)[:, None]
    rpad = (jnp.arange(RP - RH, dtype=jnp.int32) * 997) % N
    i0 = (boff + edges[:, 0][None, :]).reshape(-1)
    i1 = (boff + edges[:, 1][None, :]).reshape(-1)

    acts = []
    for h in range(2):
        i0h = jnp.concatenate([lax.dynamic_slice(i0, (h * RH,), (RH,)), rpad])
        i1h = jnp.concatenate([lax.dynamic_slice(i1, (h * RH,), (RH,)), rpad])
        g2 = _sc_pair_gather(xs2, xd2, i0h, i1h)
        acts.append(_tc_head(g2, b1, W2, b2, Wmu, bmu))
    act = jnp.concatenate(acts, axis=0)
    return act.reshape(NB, P)


# speedup vs baseline: 1.8271x; 1.0292x over previous
import functools

import jax
import jax.numpy as jnp
from jax import lax
from jax.experimental import pallas as pl
from jax.experimental.pallas import tpu as pltpu
from jax.experimental.pallas import tpu_sc as plsc

N = 10000
NP = 10240
F = 256
FH = 128
E = 160000
P = 8000
NB = 10
A = 1000
R = NB * P
LOW, HIGH = 0.0, 480.0

NC, NS = 2, 16
NW = NC * NS
CH = 128
CP = 128
E2 = 163840
EC2 = E2 // CH
ECT_B = EC2 // NS
RH = R // 2
RP = 40960
RC2 = RP // CP
RCT = RC2 // NW
ECT_A = EC2 // NW
ROWS_PER_TILE = NP // NS

_mesh = plsc.VectorSubcoreMesh(core_axis_name="c", subcore_axis_name="s")


@functools.partial(
    pl.kernel, mesh=_mesh,
    out_type=jax.ShapeDtypeStruct((NC, NP, FH), jnp.float32),
    scratch_types=[pltpu.VMEM((ECT_A, CH), jnp.int32),
                   pltpu.VMEM((CH, FH), jnp.float32),
                   pltpu.VMEM_SHARED((NP, FH), jnp.float32),
                   pltpu.SemaphoreType.DMA],
)
def _sc_deg(dst2_hbm, ones_hbm, zeros_hbm, out_hbm, idx_all, ones_v, acc_sh, sem):
    c = lax.axis_index("c")
    s = lax.axis_index("s")
    wid = s * NC + c
    pltpu.sync_copy(ones_hbm, ones_v)
    sl = pl.ds(s * ROWS_PER_TILE, ROWS_PER_TILE)
    pltpu.sync_copy(zeros_hbm, acc_sh.at[sl])
    pltpu.sync_copy(dst2_hbm.at[pl.ds(wid * ECT_A, ECT_A)], idx_all)
    plsc.subcore_barrier()

    @pl.loop(0, ECT_A)
    def _(k):
        pltpu.sync_copy(ones_v, acc_sh.at[idx_all.at[k]], add=True)

    plsc.subcore_barrier()
    pltpu.sync_copy(acc_sh.at[sl], out_hbm.at[c].at[sl])


@functools.partial(
    pl.kernel, mesh=_mesh,
    out_type=jax.ShapeDtypeStruct((NC, NP, FH), jnp.float32),
    scratch_types=[pltpu.VMEM((ECT_B * CH,), jnp.int32),
                   pltpu.VMEM((ECT_B, CH), jnp.int32),
                   pltpu.VMEM((CH, FH), jnp.float32),
                   pltpu.VMEM_SHARED((NP, FH), jnp.float32),
                   pltpu.SemaphoreType.DMA],
)
def _sc_gcn_agg(src2_hbm, dst2_hbm, xsp_hbm, zeros_hbm, out_hbm,
                sidx_all, didx_all, rb0, acc_sh, g0):
    c = lax.axis_index("c")
    s = lax.axis_index("s")
    sl = pl.ds(s * ROWS_PER_TILE, ROWS_PER_TILE)
    pltpu.sync_copy(zeros_hbm, acc_sh.at[sl])
    pltpu.sync_copy(src2_hbm.at[pl.ds(s * ECT_B * CH, ECT_B * CH)], sidx_all)
    pltpu.sync_copy(dst2_hbm.at[pl.ds(s * ECT_B, ECT_B)], didx_all)
    plsc.subcore_barrier()
    tbl = xsp_hbm.at[c]

    @pl.loop(0, ECT_B)
    def _(k):
        pltpu.async_copy(tbl.at[sidx_all.at[pl.ds(k * CH, CH)]], rb0, g0).wait()
        pltpu.sync_copy(rb0, acc_sh.at[didx_all.at[k]], add=True)

    plsc.subcore_barrier()
    pltpu.sync_copy(acc_sh.at[sl], out_hbm.at[c].at[sl])


@functools.partial(
    pl.kernel, mesh=_mesh,
    out_type=jax.ShapeDtypeStruct((2, RP, F), jnp.float32),
    scratch_types=[pltpu.VMEM((RCT * CP,), jnp.int32),
                   pltpu.VMEM((RCT * CP,), jnp.int32),
                   pltpu.VMEM((CP, F), jnp.float32),
                   pltpu.VMEM((CP, F), jnp.float32),
                   pltpu.SemaphoreType.DMA,
                   pltpu.SemaphoreType.DMA],
)
def _sc_pair_gather(xs2_hbm, xd2_hbm, i02_hbm, i12_hbm, out_hbm,
                    i0v, i1v, ra, rb, g0, g1):
    c = lax.axis_index("c")
    s = lax.axis_index("s")
    wid = s * NC + c
    base = wid * RCT * CP
    pltpu.sync_copy(i02_hbm.at[pl.ds(base, RCT * CP)], i0v)
    pltpu.sync_copy(i12_hbm.at[pl.ds(base, RCT * CP)], i1v)

    @pl.loop(0, RCT)
    def _(k):
        rows = pl.ds(base + k * CP, CP)
        isl = pl.ds(k * CP, CP)
        cp0 = pltpu.async_copy(xs2_hbm.at[i0v.at[isl]], ra, g0)
        cp1 = pltpu.async_copy(xd2_hbm.at[i1v.at[isl]], rb, g1)
        cp0.wait()
        cp1.wait()
        pltpu.sync_copy(ra, out_hbm.at[0].at[rows])
        pltpu.sync_copy(rb, out_hbm.at[1].at[rows])


def _tc_scale_split(state, Wg, degp):
    BLK = 1000

    def body(st_ref, wg_ref, dg_ref, out_ref):
        xw = jnp.dot(st_ref[...], wg_ref[...], preferred_element_type=jnp.float32)
        deg = dg_ref[0][:, 0:1] + dg_ref[1][:, 0:1] + 1.0
        dinv = lax.rsqrt(deg)
        xs = xw * dinv
        out_ref[0] = xs[:, :FH]
        out_ref[1] = xs[:, FH:]

    return pl.pallas_call(
        body,
        grid=(N // BLK,),
        in_specs=[pl.BlockSpec((BLK, F), lambda i: (i, 0)),
                  pl.BlockSpec((F, F), lambda i: (0, 0)),
                  pl.BlockSpec((NC, BLK, FH), lambda i: (0, i, 0))],
        out_specs=pl.BlockSpec((NC, BLK, FH), lambda i: (0, i, 0)),
        out_shape=jax.ShapeDtypeStruct((NC, NP, FH), jnp.float32),
    )(state, Wg, degp)


def _tc_node_mlp_in(accp, xsp, degp, state, bg, W1t, W1b):
    BLK = 1000

    def body(ac_ref, xs_ref, dg_ref, st_ref, bg_ref, w1t_ref, w1b_ref,
             o1_ref, o2_ref):
        acc = jnp.concatenate([ac_ref[0], ac_ref[1]], axis=1)
        xs = jnp.concatenate([xs_ref[0], xs_ref[1]], axis=1)
        deg = dg_ref[0][:, 0:1] + dg_ref[1][:, 0:1] + 1.0
        dinv = lax.rsqrt(deg)
        gcn = (acc + xs) * dinv + bg_ref[...]
        x = jnp.maximum(gcn, 0.0) + st_ref[...]
        o1_ref[...] = jnp.dot(x, w1t_ref[...], preferred_element_type=jnp.float32)
        o2_ref[...] = jnp.dot(x, w1b_ref[...], preferred_element_type=jnp.float32)

    return pl.pallas_call(
        body,
        grid=(N // BLK,),
        in_specs=[pl.BlockSpec((NC, BLK, FH), lambda i: (0, i, 0)),
                  pl.BlockSpec((NC, BLK, FH), lambda i: (0, i, 0)),
                  pl.BlockSpec((NC, BLK, FH), lambda i: (0, i, 0)),
                  pl.BlockSpec((BLK, F), lambda i: (i, 0)),
                  pl.BlockSpec((F,), lambda i: (0,)),
                  pl.BlockSpec((F, F), lambda i: (0, 0)),
                  pl.BlockSpec((F, F), lambda i: (0, 0))],
        out_specs=[pl.BlockSpec((BLK, F), lambda i: (i, 0)),
                   pl.BlockSpec((BLK, F), lambda i: (i, 0))],
        out_shape=[jax.ShapeDtypeStruct((N, F), jnp.float32),
                   jax.ShapeDtypeStruct((N, F), jnp.float32)],
    )(accp, xsp, degp, state, bg, W1t, W1b)


def _tc_head(g2, b1, W2, b2, Wmu, bmu):
    BLK = 2000

    def body(g_ref, b1_ref, w2_ref, b2_ref, wmu_ref, bmu_ref, o_ref):
        h = g_ref[0] + g_ref[1] + b1_ref[...]
        h = jnp.where(h > 0, h, 0.01 * h)
        h2 = jnp.dot(h, w2_ref[...], preferred_element_type=jnp.float32) + b2_ref[...]
        h2 = jnp.where(h2 > 0, h2, 0.01 * h2)
        m = jnp.dot(h2, wmu_ref[...], preferred_element_type=jnp.float32) + bmu_ref[...]
        mu = jax.nn.softplus(m)
        act = (jnp.tanh(mu) + 1.0) * (0.5 * (HIGH - LOW)) + LOW
        o_ref[...] = jnp.clip(act, LOW, HIGH)

    return pl.pallas_call(
        body,
        grid=(RH // BLK,),
        in_specs=[pl.BlockSpec((2, BLK, F), lambda i: (0, i, 0)),
                  pl.BlockSpec((F,), lambda i: (0,)),
                  pl.BlockSpec((F, F), lambda i: (0, 0)),
                  pl.BlockSpec((F,), lambda i: (0,)),
                  pl.BlockSpec((F, 1), lambda i: (0, 0)),
                  pl.BlockSpec((1,), lambda i: (0,))],
        out_specs=pl.BlockSpec((BLK, 1), lambda i: (i, 0)),
        out_shape=jax.ShapeDtypeStruct((RH, 1), jnp.float32),
    )(g2, b1, W2, b2, Wmu, bmu)


def kernel(state, edge_index, edges, deterministic,
           Wg, bg, W1, b1, W2, b2, Wmu, bmu, Wsig, bsig):
    del deterministic, Wsig, bsig
    pad = E2 - E
    spread = jnp.arange(pad, dtype=jnp.int32)
    src2 = jnp.concatenate([edge_index[0], spread % N])
    dst2 = jnp.concatenate([edge_index[1], N + (spread % (NP - N))])
    onesH = jnp.ones((CH, FH), jnp.float32)
    zerosH = jnp.zeros((ROWS_PER_TILE, FH), jnp.float32)

    degp = _sc_deg(dst2.reshape(EC2, CH), onesH, zerosH)
    xsp = _tc_scale_split(state, Wg, degp)
    accp = _sc_gcn_agg(src2, dst2.reshape(EC2, CH), xsp, zerosH)
    xs2, xd2 = _tc_node_mlp_in(accp, xsp, degp, state, bg,
                               W1[:F, :], W1[F:, :])

    boff = (jnp.arange(NB, dtype=jnp.int32) * A)[:, None]
    rpad = (jnp.arange(RP - RH, dtype=jnp.int32) * 997) % N
    i0 = (boff + edges[:, 0][None, :]).reshape(-1)
    i1 = (boff + edges[:, 1][None, :]).reshape(-1)

    acts = []
    for h in range(2):
        i0h = jnp.concatenate([lax.dynamic_slice(i0, (h * RH,), (RH,)), rpad])
        i1h = jnp.concatenate([lax.dynamic_slice(i1, (h * RH,), (RH,)), rpad])
        g2 = _sc_pair_gather(xs2, xd2, i0h, i1h)
        acts.append(_tc_head(g2, b1, W2, b2, Wmu, bmu))
    act = jnp.concatenate(acts, axis=0)
    return act.reshape(NB, P)


# speedup vs baseline: 1.8343x; 1.0039x over previous
import functools

import jax
import jax.numpy as jnp
from jax import lax
from jax.experimental import pallas as pl
from jax.experimental.pallas import tpu as pltpu
from jax.experimental.pallas import tpu_sc as plsc

N = 10000
NP = 10240
F = 256
FH = 128
E = 160000
P = 8000
NB = 10
A = 1000
R = NB * P
LOW, HIGH = 0.0, 480.0

NC, NS = 2, 16
NW = NC * NS
CH = 128
CP = 128
E2 = 163840
EC2 = E2 // CH
ECT_B = EC2 // NS
RH = R // 2
RP = 40960
RC2 = RP // CP
RCT = RC2 // NW
ECT_A = EC2 // NW
ROWS_PER_TILE = NP // NS

_mesh = plsc.VectorSubcoreMesh(core_axis_name="c", subcore_axis_name="s")


@functools.partial(
    pl.kernel, mesh=_mesh,
    out_type=jax.ShapeDtypeStruct((NC, NP, FH), jnp.float32),
    scratch_types=[pltpu.VMEM((ECT_A, CH), jnp.int32),
                   pltpu.VMEM((CH, FH), jnp.float32),
                   pltpu.VMEM_SHARED((NP, FH), jnp.float32),
                   pltpu.SemaphoreType.DMA],
)
def _sc_deg(dst2_hbm, ones_hbm, zeros_hbm, out_hbm, idx_all, ones_v, acc_sh, sem):
    c = lax.axis_index("c")
    s = lax.axis_index("s")
    wid = s * NC + c
    pltpu.sync_copy(ones_hbm, ones_v)
    sl = pl.ds(s * ROWS_PER_TILE, ROWS_PER_TILE)
    pltpu.sync_copy(zeros_hbm, acc_sh.at[sl])
    pltpu.sync_copy(dst2_hbm.at[pl.ds(wid * ECT_A, ECT_A)], idx_all)
    plsc.subcore_barrier()

    @pl.loop(0, ECT_A)
    def _(k):
        pltpu.sync_copy(ones_v, acc_sh.at[idx_all.at[k]], add=True)

    plsc.subcore_barrier()
    pltpu.sync_copy(acc_sh.at[sl], out_hbm.at[c].at[sl])


@functools.partial(
    pl.kernel, mesh=_mesh,
    out_type=jax.ShapeDtypeStruct((NC, NP, FH), jnp.float32),
    scratch_types=[pltpu.VMEM((ECT_B * CH,), jnp.int32),
                   pltpu.VMEM((ECT_B, CH), jnp.int32),
                   pltpu.VMEM((CH, FH), jnp.float32),
                   pltpu.VMEM_SHARED((NP, FH), jnp.float32),
                   pltpu.SemaphoreType.DMA],
)
def _sc_gcn_agg(src2_hbm, dst2_hbm, xsp_hbm, zeros_hbm, out_hbm,
                sidx_all, didx_all, rb0, acc_sh, g0):
    c = lax.axis_index("c")
    s = lax.axis_index("s")
    sl = pl.ds(s * ROWS_PER_TILE, ROWS_PER_TILE)
    pltpu.sync_copy(zeros_hbm, acc_sh.at[sl])
    pltpu.sync_copy(src2_hbm.at[pl.ds(s * ECT_B * CH, ECT_B * CH)], sidx_all)
    pltpu.sync_copy(dst2_hbm.at[pl.ds(s * ECT_B, ECT_B)], didx_all)
    plsc.subcore_barrier()
    tbl = xsp_hbm.at[c]

    @pl.loop(0, ECT_B)
    def _(k):
        pltpu.async_copy(tbl.at[sidx_all.at[pl.ds(k * CH, CH)]], rb0, g0).wait()
        pltpu.sync_copy(rb0, acc_sh.at[didx_all.at[k]], add=True)

    plsc.subcore_barrier()
    pltpu.sync_copy(acc_sh.at[sl], out_hbm.at[c].at[sl])


@functools.partial(
    pl.kernel, mesh=_mesh,
    out_type=jax.ShapeDtypeStruct((2, RP, F), jnp.float32),
    scratch_types=[pltpu.VMEM((RCT * CP,), jnp.int32),
                   pltpu.VMEM((RCT * CP,), jnp.int32),
                   pltpu.VMEM((CP, F), jnp.float32),
                   pltpu.VMEM((CP, F), jnp.float32),
                   pltpu.SemaphoreType.DMA,
                   pltpu.SemaphoreType.DMA],
)
def _sc_pair_gather(xs2_hbm, xd2_hbm, i02_hbm, i12_hbm, out_hbm,
                    i0v, i1v, ra, rb, g0, g1):
    c = lax.axis_index("c")
    s = lax.axis_index("s")
    wid = s * NC + c
    base = wid * RCT * CP
    pltpu.sync_copy(i02_hbm.at[pl.ds(base, RCT * CP)], i0v)
    pltpu.sync_copy(i12_hbm.at[pl.ds(base, RCT * CP)], i1v)

    @pl.loop(0, RCT)
    def _(k):
        rows = pl.ds(base + k * CP, CP)
        isl = pl.ds(k * CP, CP)
        cp0 = pltpu.async_copy(xs2_hbm.at[i0v.at[isl]], ra, g0)
        cp1 = pltpu.async_copy(xd2_hbm.at[i1v.at[isl]], rb, g1)
        cp0.wait()
        pltpu.sync_copy(ra, out_hbm.at[0].at[rows])
        cp1.wait()
        pltpu.sync_copy(rb, out_hbm.at[1].at[rows])


def _tc_scale_split(state, Wg, degp):
    BLK = 1000

    def body(st_ref, wg_ref, dg_ref, out_ref):
        xw = jnp.dot(st_ref[...], wg_ref[...], preferred_element_type=jnp.float32)
        deg = dg_ref[0][:, 0:1] + dg_ref[1][:, 0:1] + 1.0
        dinv = lax.rsqrt(deg)
        xs = xw * dinv
        out_ref[0] = xs[:, :FH]
        out_ref[1] = xs[:, FH:]

    return pl.pallas_call(
        body,
        grid=(N // BLK,),
        in_specs=[pl.BlockSpec((BLK, F), lambda i: (i, 0)),
                  pl.BlockSpec((F, F), lambda i: (0, 0)),
                  pl.BlockSpec((NC, BLK, FH), lambda i: (0, i, 0))],
        out_specs=pl.BlockSpec((NC, BLK, FH), lambda i: (0, i, 0)),
        out_shape=jax.ShapeDtypeStruct((NC, NP, FH), jnp.float32),
    )(state, Wg, degp)


def _tc_node_mlp_in(accp, xsp, degp, state, bg, W1t, W1b):
    BLK = 1000

    def body(ac_ref, xs_ref, dg_ref, st_ref, bg_ref, w1t_ref, w1b_ref,
             o1_ref, o2_ref):
        acc = jnp.concatenate([ac_ref[0], ac_ref[1]], axis=1)
        xs = jnp.concatenate([xs_ref[0], xs_ref[1]], axis=1)
        deg = dg_ref[0][:, 0:1] + dg_ref[1][:, 0:1] + 1.0
        dinv = lax.rsqrt(deg)
        gcn = (acc + xs) * dinv + bg_ref[...]
        x = jnp.maximum(gcn, 0.0) + st_ref[...]
        o1_ref[...] = jnp.dot(x, w1t_ref[...], preferred_element_type=jnp.float32)
        o2_ref[...] = jnp.dot(x, w1b_ref[...], preferred_element_type=jnp.float32)

    return pl.pallas_call(
        body,
        grid=(N // BLK,),
        in_specs=[pl.BlockSpec((NC, BLK, FH), lambda i: (0, i, 0)),
                  pl.BlockSpec((NC, BLK, FH), lambda i: (0, i, 0)),
                  pl.BlockSpec((NC, BLK, FH), lambda i: (0, i, 0)),
                  pl.BlockSpec((BLK, F), lambda i: (i, 0)),
                  pl.BlockSpec((F,), lambda i: (0,)),
                  pl.BlockSpec((F, F), lambda i: (0, 0)),
                  pl.BlockSpec((F, F), lambda i: (0, 0))],
        out_specs=[pl.BlockSpec((BLK, F), lambda i: (i, 0)),
                   pl.BlockSpec((BLK, F), lambda i: (i, 0))],
        out_shape=[jax.ShapeDtypeStruct((N, F), jnp.float32),
                   jax.ShapeDtypeStruct((N, F), jnp.float32)],
    )(accp, xsp, degp, state, bg, W1t, W1b)


def _tc_head(g2, b1, W2, b2, Wmu, bmu):
    BLK = 2000

    def body(g_ref, b1_ref, w2_ref, b2_ref, wmu_ref, bmu_ref, o_ref):
        h = g_ref[0] + g_ref[1] + b1_ref[...]
        h = jnp.where(h > 0, h, 0.01 * h)
        h2 = jnp.dot(h, w2_ref[...], preferred_element_type=jnp.float32) + b2_ref[...]
        h2 = jnp.where(h2 > 0, h2, 0.01 * h2)
        m = jnp.dot(h2, wmu_ref[...], preferred_element_type=jnp.float32) + bmu_ref[...]
        mu = jax.nn.softplus(m)
        act = (jnp.tanh(mu) + 1.0) * (0.5 * (HIGH - LOW)) + LOW
        o_ref[...] = jnp.clip(act, LOW, HIGH)

    return pl.pallas_call(
        body,
        grid=(RH // BLK,),
        in_specs=[pl.BlockSpec((2, BLK, F), lambda i: (0, i, 0)),
                  pl.BlockSpec((F,), lambda i: (0,)),
                  pl.BlockSpec((F, F), lambda i: (0, 0)),
                  pl.BlockSpec((F,), lambda i: (0,)),
                  pl.BlockSpec((F, 1), lambda i: (0, 0)),
                  pl.BlockSpec((1,), lambda i: (0,))],
        out_specs=pl.BlockSpec((BLK, 1), lambda i: (i, 0)),
        out_shape=jax.ShapeDtypeStruct((RH, 1), jnp.float32),
    )(g2, b1, W2, b2, Wmu, bmu)


def kernel(state, edge_index, edges, deterministic,
           Wg, bg, W1, b1, W2, b2, Wmu, bmu, Wsig, bsig):
    del deterministic, Wsig, bsig
    pad = E2 - E
    spread = jnp.arange(pad, dtype=jnp.int32)
    src2 = jnp.concatenate([edge_index[0], spread % N])
    dst2 = jnp.concatenate([edge_index[1], N + (spread % (NP - N))])
    onesH = jnp.ones((CH, FH), jnp.float32)
    zerosH = jnp.zeros((ROWS_PER_TILE, FH), jnp.float32)

    degp = _sc_deg(dst2.reshape(EC2, CH), onesH, zerosH)
    xsp = _tc_scale_split(state, Wg, degp)
    accp = _sc_gcn_agg(src2, dst2.reshape(EC2, CH), xsp, zerosH)
    xs2, xd2 = _tc_node_mlp_in(accp, xsp, degp, state, bg,
                               W1[:F, :], W1[F:, :])

    boff = (jnp.arange(NB, dtype=jnp.int32) * A)[:, None]
    rpad = (jnp.arange(RP - RH, dtype=jnp.int32) * 997) % N
    i0 = (boff + edges[:, 0][None, :]).reshape(-1)
    i1 = (boff + edges[:, 1][None, :]).reshape(-1)

    acts = []
    for h in range(2):
        i0h = jnp.concatenate([lax.dynamic_slice(i0, (h * RH,), (RH,)), rpad])
        i1h = jnp.concatenate([lax.dynamic_slice(i1, (h * RH,), (RH,)), rpad])
        g2 = _sc_pair_gather(xs2, xd2, i0h, i1h)
        acts.append(_tc_head(g2, b1, W2, b2, Wmu, bmu))
    act = jnp.concatenate(acts, axis=0)
    return act.reshape(NB, P)
